# baseline (speedup 1.0000x reference)
import functools

import jax
import jax.numpy as jnp
from jax import lax
from jax.experimental import pallas as pl
from jax.experimental.pallas import tpu as pltpu
from jax.experimental.pallas import tpu_sc as plsc

_F32 = jnp.float32
_I32 = jnp.int32


def _sc_geometry():
    try:
        info = plsc.get_sparse_core_info()
        return info.num_cores, info.num_subcores, info.num_lanes
    except Exception:
        return 2, 16, 16


def _pick_chunk(epw, cap=128):
    for c in range(cap, 7, -8):
        if epw % c == 0:
            return c
    return 8


def _row_partition(N, NS):
    npad = -(-N // (NS * 8)) * (NS * 8)
    rps = npad // NS
    last = N - (NS - 1) * rps
    return npad, rps, last


def _sc_segsum(z, src_p, dst_p, EPW, CH):
    N, H = z.shape
    NC, NS, LN = _sc_geometry()
    NIT = EPW // CH
    NPAD, RPS, LASTR = _row_partition(N, NS)
    ZR = 8
    mesh = plsc.VectorSubcoreMesh(core_axis_name="c", subcore_axis_name="s")

    NB = 8

    @functools.partial(
        pl.kernel,
        mesh=mesh,
        out_type=jax.ShapeDtypeStruct((NC, N, H), _F32),
        scratch_types=[
            pltpu.VMEM((NB, CH), _I32),
            pltpu.VMEM((NB, CH), _I32),
            pltpu.VMEM((NB, CH, H), _F32),
            pltpu.VMEM((ZR, H), _F32),
            pltpu.VMEM_SHARED((NPAD, H), _F32),
        ] + [pltpu.SemaphoreType.DMA] * (3 * NB + 1),
    )
    def k(z_hbm, src_hbm, dst_hbm, out_hbm, srcb, dstb, rows, zero_v, acc,
          *sems):
        semi = sems[0:NB]
        semg = sems[NB:2 * NB]
        sems_ = sems[2 * NB:3 * NB]
        semz = sems[3 * NB]
        c = lax.axis_index("c")
        s = lax.axis_index("s")
        wid = s * NC + c
        ebase = wid * EPW
        zvec = jnp.zeros((LN,), _F32)

        def zrow(i, carry):
            for j in range(H // LN):
                zero_v[i, pl.ds(j * LN, LN)] = zvec
            return carry

        lax.fori_loop(0, ZR, zrow, 0)
        rbase = s * RPS

        def fetch_idx(i, u):
            pltpu.async_copy(src_hbm.at[pl.ds(ebase + i * CH, CH)],
                             srcb.at[u], semi[u])
            pltpu.async_copy(dst_hbm.at[pl.ds(ebase + i * CH, CH)],
                             dstb.at[u], semi[u])

        def gather(u):
            pltpu.make_async_copy(src_hbm.at[pl.ds(0, CH)], srcb.at[u],
                                  semi[u]).wait()
            pltpu.make_async_copy(dst_hbm.at[pl.ds(0, CH)], dstb.at[u],
                                  semi[u]).wait()
            pltpu.async_copy(z_hbm.at[srcb.at[u]], rows.at[u], semg[u])

        def scatter(u):
            pltpu.make_async_copy(z_hbm.at[pl.ds(0, CH)], rows.at[u],
                                  semg[u]).wait()
            pltpu.async_copy(rows.at[u], acc.at[dstb.at[u]], sems_[u],
                             add=True)

        def wait_scat(u):
            pltpu.make_async_copy(z_hbm.at[pl.ds(0, CH)], rows.at[u],
                                  sems_[u]).wait()

        for u in range(NB):
            if u < NIT:
                fetch_idx(u, u)
        for u in range(NB):
            if u < NIT:
                gather(u)

        def zacc(i, carry):
            pltpu.async_copy(zero_v, acc.at[pl.ds(rbase + i * ZR, ZR)], semz)
            return carry

        lax.fori_loop(0, RPS // ZR, zacc, 0)

        def zdrain(i, carry):
            pltpu.make_async_copy(z_hbm.at[pl.ds(0, ZR)], zero_v, semz).wait()
            return carry

        lax.fori_loop(0, RPS // ZR, zdrain, 0)
        plsc.subcore_barrier()

        NFULL = NIT // NB

        def body(j, carry):
            base = j * NB
            for u in range(NB):
                scatter(u)
            for u in range(NB):
                nxt = base + NB + u

                @pl.when(nxt < NIT)
                def _():
                    wait_scat(u)
                    fetch_idx(nxt, u)
                    gather(u)

            return carry

        lax.fori_loop(0, NFULL, body, 0)
        for u in range(NIT - NFULL * NB):
            scatter(u)
        for u in range(min(NB, NIT)):
            wait_scat(u)
        plsc.subcore_barrier()

        @pl.when(s < NS - 1)
        def _full():
            pltpu.sync_copy(acc.at[pl.ds(rbase, RPS)],
                            out_hbm.at[c, pl.ds(rbase, RPS)])

        @pl.when(s == NS - 1)
        def _tail():
            pltpu.sync_copy(acc.at[pl.ds((NS - 1) * RPS, LASTR)],
                            out_hbm.at[c, pl.ds((NS - 1) * RPS, LASTR)])

    return k(z, src_p, dst_p)


def _sc_edgeprep_impl(edge_attr, dst_p, N, H, EPW, CH):
    E, ED = edge_attr.shape
    NC, NS, LN = _sc_geometry()
    NIT = EPW // CH
    NPAD, RPS, LASTR = _row_partition(N, NS)
    ZR = 8
    mesh = plsc.VectorSubcoreMesh(core_axis_name="c", subcore_axis_name="s")

    NB = 2

    @functools.partial(
        pl.kernel,
        mesh=mesh,
        out_type=jax.ShapeDtypeStruct((NC, N, H), _F32),
        scratch_types=[
            pltpu.VMEM((NB, CH), _I32),
            pltpu.VMEM((NB, CH), _I32),
            pltpu.VMEM((NB, CH, ED), _F32),
            pltpu.VMEM((NB, CH, H), _F32),
            pltpu.VMEM((ZR, H), _F32),
            pltpu.VMEM_SHARED((NPAD, H), _F32),
        ] + [pltpu.SemaphoreType.DMA] * (2 * NB + 1),
    )
    def k(ea_hbm, dst_hbm, aux_out, dst_v, dsc_v, ea_v, rows_v, zero_v, acc,
          *sems):
        semi = sems[0:NB]
        sems_ = sems[NB:2 * NB]
        semz = sems[2 * NB]
        c = lax.axis_index("c")
        s = lax.axis_index("s")
        wid = s * NC + c
        ebase = wid * EPW
        zvec = jnp.zeros((LN,), _F32)
        lane = lax.iota(_I32, LN)
        onehot = jnp.where(lane == 0, jnp.float32(1.0), jnp.float32(0.0))

        def init_zero(i, carry):
            for j in range(H // LN):
                zero_v[i, pl.ds(j * LN, LN)] = zvec
            return carry

        lax.fori_loop(0, ZR, init_zero, 0)

        def init_rows(i, carry):
            for u in range(NB):
                rows_v[u, i, pl.ds(LN, LN)] = onehot
                for j in range(2, H // LN):
                    rows_v[u, i, pl.ds(j * LN, LN)] = zvec
            return carry

        lax.fori_loop(0, CH, init_rows, 0)
        rbase = s * RPS

        def fetch(i, u):
            b = ebase + i * CH
            pltpu.async_copy(dst_hbm.at[pl.ds(b, CH)], dst_v.at[u], semi[u])
            pltpu.async_copy(ea_hbm.at[pl.ds(b, CH)], ea_v.at[u], semi[u])

        def wait_fetch(u):
            pltpu.make_async_copy(dst_hbm.at[pl.ds(0, CH)], dst_v.at[u],
                                  semi[u]).wait()
            pltpu.make_async_copy(ea_hbm.at[pl.ds(0, CH)], ea_v.at[u],
                                  semi[u]).wait()

        def wait_scat(u):
            pltpu.make_async_copy(z_dummy.at[pl.ds(0, CH)], rows_v.at[u],
                                  sems_[u]).wait()

        z_dummy = aux_out.at[0]

        for u in range(NB):
            fetch(u, u)

        def zacc(i, carry):
            pltpu.async_copy(zero_v, acc.at[pl.ds(rbase + i * ZR, ZR)], semz)
            return carry

        lax.fori_loop(0, RPS // ZR, zacc, 0)

        def zdrain(i, carry):
            pltpu.make_async_copy(z_dummy.at[pl.ds(0, ZR)], zero_v,
                                  semz).wait()
            return carry

        lax.fori_loop(0, RPS // ZR, zdrain, 0)
        plsc.subcore_barrier()

        def stage(i, u):
            @pl.when(i > 0)
            def _():
                wait_scat(u)

            wait_fetch(u)

            def repack(r, carry2):
                rows_v[u, r, pl.ds(0, LN)] = ea_v[u, r, pl.ds(0, LN)]
                return carry2

            lax.fori_loop(0, CH, repack, 0)
            for g in range(CH // LN):
                dsc_v[u, pl.ds(g * LN, LN)] = dst_v[u, pl.ds(g * LN, LN)]
            pltpu.async_copy(rows_v.at[u], acc.at[dsc_v.at[u]], sems_[u],
                             add=True)

        def body(i, carry):
            for u in range(NB):
                idx = i * NB + u
                stage(i, u)

                @pl.when(idx + NB < NIT)
                def _():
                    fetch(idx + NB, u)

            return carry

        lax.fori_loop(0, NIT // NB, body, 0)
        for u in range(NIT - (NIT // NB) * NB):
            stage(NIT // NB, u)
        for u in range(min(NB, NIT)):
            wait_scat(u)
        plsc.subcore_barrier()

        @pl.when(s < NS - 1)
        def _full():
            pltpu.sync_copy(acc.at[pl.ds(rbase, RPS)],
                            aux_out.at[c, pl.ds(rbase, RPS)])

        @pl.when(s == NS - 1)
        def _tail():
            pltpu.sync_copy(acc.at[pl.ds((NS - 1) * RPS, LASTR)],
                            aux_out.at[c, pl.ds((NS - 1) * RPS, LASTR)])

    return k(edge_attr, dst_p)


def _dotT(a, w):
    return lax.dot_general(a, w, (((1,), (1,)), ((), ())),
                           preferred_element_type=_F32)


def _tc_auxred(aux):
    NC, N, H = aux.shape

    def body(a_ref, o_ref):
        o_ref[...] = a_ref[0, :, 0:32] + a_ref[1, :, 0:32]

    return pl.pallas_call(
        body, out_shape=jax.ShapeDtypeStruct((N, 32), _F32))(aux)


def _tc_prep(batch2d, x, nW0, vn_emb, Bsz):
    _, N = batch2d.shape
    H = nW0.shape[0]

    def body(batch_ref, x_ref, w_ref, vne_ref, P_ref, Pn_ref, z_ref, vn_ref):
        iota = lax.broadcasted_iota(_I32, (Bsz, N), 0)
        P = (iota == batch_ref[...]).astype(_F32)
        counts = jnp.sum(P, axis=1, keepdims=True)
        Pn = P / jnp.maximum(counts, 1.0)
        P_ref[...] = P
        Pn_ref[...] = Pn
        vn0 = jnp.broadcast_to(vne_ref[...], (Bsz, H))
        vn_ref[...] = vn0
        z_ref[...] = _dotT(x_ref[...], w_ref[...]) + lax.dot_general(
            P, vn0, (((0,), (0,)), ((), ())), preferred_element_type=_F32)

    return pl.pallas_call(
        body,
        out_shape=(
            jax.ShapeDtypeStruct((Bsz, N), _F32),
            jax.ShapeDtypeStruct((Bsz, N), _F32),
            jax.ShapeDtypeStruct((N, H), _F32),
            jax.ShapeDtypeStruct((Bsz, H), _F32),
        ),
    )(batch2d, x, nW0, vn_emb)


def _tc_layer(Ap, ead, vn, P, Pn, eW, beff, W1, b1, W2, b2,
              vW1, vb1, vW2, vb2, nW_next):
    NC, N, H = Ap.shape
    Bsz = P.shape[0]

    def body(Ap_ref, ead_ref, vn_ref, P_ref, Pn_ref, eW_ref, beff_ref,
             W1_ref, b1_ref, W2_ref, b2_ref, vW1_ref, vb1_ref, vW2_ref,
             vb2_ref, nWn_ref, z_ref, vno_ref):
        A = Ap_ref[0] + Ap_ref[1]
        ead = ead_ref[...]
        EA = ead[:, 0:16]
        deg = ead[:, 16:17]
        agg = A + _dotT(EA, eW_ref[...]) + deg * beff_ref[...]
        t = jnp.maximum(_dotT(agg, W1_ref[...]) + b1_ref[...], 0.0)
        h = jnp.maximum(_dotT(t, W2_ref[...]) + b2_ref[...], 0.0)
        pooled = jnp.dot(Pn_ref[...], h, preferred_element_type=_F32)
        q = jnp.maximum(_dotT(pooled, vW1_ref[...]) + vb1_ref[...], 0.0)
        vnu = jnp.maximum(_dotT(q, vW2_ref[...]) + vb2_ref[...], 0.0)
        vn_new = vn_ref[...] + vnu
        vno_ref[...] = vn_new
        z_ref[...] = _dotT(h, nWn_ref[...]) + lax.dot_general(
            P_ref[...], vn_new, (((0,), (0,)), ((), ())),
            preferred_element_type=_F32)

    return pl.pallas_call(
        body,
        out_shape=(
            jax.ShapeDtypeStruct((N, H), _F32),
            jax.ShapeDtypeStruct((Bsz, H), _F32),
        ),
    )(Ap, ead, vn, P, Pn, eW, beff, W1, b1, W2, b2, vW1, vb1, vW2, vb2,
      nW_next)


def _tc_last(Ap, ead, Pn, eW, beff, W1, b1, W2, b2, fcW, fcb):
    NC, N, H = Ap.shape
    Bsz = Pn.shape[0]
    OUT = fcW.shape[0]

    def body(Ap_ref, ead_ref, Pn_ref, eW_ref, beff_ref, W1_ref,
             b1_ref, W2_ref, b2_ref, fcW_ref, fcb_ref, o_ref):
        A = Ap_ref[0] + Ap_ref[1]
        ead = ead_ref[...]
        EA = ead[:, 0:16]
        deg = ead[:, 16:17]
        agg = A + _dotT(EA, eW_ref[...]) + deg * beff_ref[...]
        t = jnp.maximum(_dotT(agg, W1_ref[...]) + b1_ref[...], 0.0)
        h = jnp.maximum(_dotT(t, W2_ref[...]) + b2_ref[...], 0.0)
        pooled = jnp.dot(Pn_ref[...], h, preferred_element_type=_F32)
        o_ref[...] = _dotT(pooled, fcW_ref[...]) + fcb_ref[...]

    return pl.pallas_call(
        body,
        out_shape=jax.ShapeDtypeStruct((Bsz, OUT), _F32),
    )(Ap, ead, Pn, eW, beff, W1, b1, W2, b2, fcW, fcb)


def kernel(x, edge_index, edge_attr, batch, node_W, node_b, edge_W, edge_b,
           mlp_W1, mlp_b1, mlp_W2, mlp_b2, vn_emb, vnmlp_W1, vnmlp_b1,
           vnmlp_W2, vnmlp_b2, fc_W, fc_b):
    N, F = x.shape
    L, H, _ = node_W.shape
    E = edge_index.shape[1]
    ED = edge_attr.shape[1]
    Bsz = 128
    NC, NS, _LN = _sc_geometry()
    NW = NC * NS
    NPAD, _RPS, _LASTR = _row_partition(N, NS)
    CH = 32
    EPW = -(-E // (NW * CH)) * CH
    pad = NW * EPW - E
    src = edge_index[0]
    dst = edge_index[1]
    src_p = jnp.concatenate([src, jnp.zeros((pad,), _I32)])
    dst_p = jnp.concatenate([dst, jnp.full((pad,), N, _I32)])
    ea_p = jnp.concatenate([edge_attr, jnp.zeros((pad, ED), _F32)])
    r2 = lambda v: v.reshape(1, -1)

    aux = _sc_edgeprep_impl(ea_p, dst_p, N, H, EPW, CH)
    ead = _tc_auxred(aux)
    P, Pn, z, vn = _tc_prep(batch.reshape(1, N), x, node_W[0], vn_emb, Bsz)

    out = None
    for l in range(L):
        Ap = _sc_segsum(z, src_p, dst_p, EPW, CH)
        beff = r2(node_b[l] + edge_b[l])
        if l + 1 < L:
            z, vn = _tc_layer(Ap, ead, vn, P, Pn, edge_W[l], beff,
                              mlp_W1[l], r2(mlp_b1[l]), mlp_W2[l],
                              r2(mlp_b2[l]), vnmlp_W1, r2(vnmlp_b1),
                              vnmlp_W2, r2(vnmlp_b2), node_W[l + 1])
        else:
            out = _tc_last(Ap, ead, Pn, edge_W[l], beff, mlp_W1[l],
                           r2(mlp_b1[l]), mlp_W2[l], r2(mlp_b2[l]), fc_W,
                           r2(fc_b))
    return out


# speedup vs baseline: 1.0166x; 1.0166x over previous
import functools

import jax
import jax.numpy as jnp
from jax import lax
from jax.experimental import pallas as pl
from jax.experimental.pallas import tpu as pltpu
from jax.experimental.pallas import tpu_sc as plsc

_F32 = jnp.float32
_I32 = jnp.int32


def _sc_geometry():
    try:
        info = plsc.get_sparse_core_info()
        return info.num_cores, info.num_subcores, info.num_lanes
    except Exception:
        return 2, 16, 16


def _pick_chunk(epw, cap=128):
    for c in range(cap, 7, -8):
        if epw % c == 0:
            return c
    return 8


def _row_partition(N, NS):
    npad = -(-N // (NS * 8)) * (NS * 8)
    rps = npad // NS
    last = N - (NS - 1) * rps
    return npad, rps, last


def _sc_segsum(z, src_p, dst_p, EPW, CH):
    N, H = z.shape
    NC, NS, LN = _sc_geometry()
    NIT = EPW // CH
    NPAD, RPS, LASTR = _row_partition(N, NS)
    ZR = 8
    mesh = plsc.VectorSubcoreMesh(core_axis_name="c", subcore_axis_name="s")

    NB = 4

    @functools.partial(
        pl.kernel,
        mesh=mesh,
        out_type=jax.ShapeDtypeStruct((NC, N, H), _F32),
        scratch_types=[
            pltpu.VMEM((NB, CH), _I32),
            pltpu.VMEM((NB, CH), _I32),
            pltpu.VMEM((NB, CH, H), _F32),
            pltpu.VMEM((ZR, H), _F32),
            pltpu.VMEM_SHARED((NPAD, H), _F32),
        ] + [pltpu.SemaphoreType.DMA] * (3 * NB + 1),
    )
    def k(z_hbm, src_hbm, dst_hbm, out_hbm, srcb, dstb, rows, zero_v, acc,
          *sems):
        semi = sems[0:NB]
        semg = sems[NB:2 * NB]
        sems_ = sems[2 * NB:3 * NB]
        semz = sems[3 * NB]
        c = lax.axis_index("c")
        s = lax.axis_index("s")
        wid = s * NC + c
        ebase = wid * EPW
        zvec = jnp.zeros((LN,), _F32)

        def zrow(i, carry):
            for j in range(H // LN):
                zero_v[i, pl.ds(j * LN, LN)] = zvec
            return carry

        lax.fori_loop(0, ZR, zrow, 0)
        rbase = s * RPS

        def fetch_idx(i, u):
            pltpu.async_copy(src_hbm.at[pl.ds(ebase + i * CH, CH)],
                             srcb.at[u], semi[u])
            pltpu.async_copy(dst_hbm.at[pl.ds(ebase + i * CH, CH)],
                             dstb.at[u], semi[u])

        def gather(u):
            pltpu.make_async_copy(src_hbm.at[pl.ds(0, CH)], srcb.at[u],
                                  semi[u]).wait()
            pltpu.make_async_copy(dst_hbm.at[pl.ds(0, CH)], dstb.at[u],
                                  semi[u]).wait()
            pltpu.async_copy(z_hbm.at[srcb.at[u]], rows.at[u], semg[u])

        def scatter(u):
            pltpu.make_async_copy(z_hbm.at[pl.ds(0, CH)], rows.at[u],
                                  semg[u]).wait()
            pltpu.async_copy(rows.at[u], acc.at[dstb.at[u]], sems_[u],
                             add=True)

        def wait_scat(u):
            pltpu.make_async_copy(z_hbm.at[pl.ds(0, CH)], rows.at[u],
                                  sems_[u]).wait()

        for u in range(NB):
            if u < NIT:
                fetch_idx(u, u)
        for u in range(NB):
            if u < NIT:
                gather(u)

        def zacc(i, carry):
            pltpu.async_copy(zero_v, acc.at[pl.ds(rbase + i * ZR, ZR)], semz)
            return carry

        lax.fori_loop(0, RPS // ZR, zacc, 0)

        def zdrain(i, carry):
            pltpu.make_async_copy(z_hbm.at[pl.ds(0, ZR)], zero_v, semz).wait()
            return carry

        lax.fori_loop(0, RPS // ZR, zdrain, 0)
        plsc.subcore_barrier()

        NFULL = NIT // NB

        def body(j, carry):
            base = j * NB
            for u in range(NB):
                scatter(u)
            for u in range(NB):
                nxt = base + NB + u

                @pl.when(nxt < NIT)
                def _():
                    wait_scat(u)
                    fetch_idx(nxt, u)
                    gather(u)

            return carry

        lax.fori_loop(0, NFULL, body, 0)
        for u in range(NIT - NFULL * NB):
            scatter(u)
        for u in range(min(NB, NIT)):
            wait_scat(u)
        plsc.subcore_barrier()

        @pl.when(s < NS - 1)
        def _full():
            pltpu.sync_copy(acc.at[pl.ds(rbase, RPS)],
                            out_hbm.at[c, pl.ds(rbase, RPS)])

        @pl.when(s == NS - 1)
        def _tail():
            pltpu.sync_copy(acc.at[pl.ds((NS - 1) * RPS, LASTR)],
                            out_hbm.at[c, pl.ds((NS - 1) * RPS, LASTR)])

    return k(z, src_p, dst_p)


def _sc_edgeprep_impl(edge_attr, dst_p, N, H, EPW, CH):
    E, ED = edge_attr.shape
    NC, NS, LN = _sc_geometry()
    NIT = EPW // CH
    NPAD, RPS, LASTR = _row_partition(N, NS)
    ZR = 8
    mesh = plsc.VectorSubcoreMesh(core_axis_name="c", subcore_axis_name="s")

    NB = 2

    @functools.partial(
        pl.kernel,
        mesh=mesh,
        out_type=jax.ShapeDtypeStruct((NC, N, H), _F32),
        scratch_types=[
            pltpu.VMEM((NB, CH), _I32),
            pltpu.VMEM((NB, CH), _I32),
            pltpu.VMEM((NB, CH, ED), _F32),
            pltpu.VMEM((NB, CH, H), _F32),
            pltpu.VMEM((ZR, H), _F32),
            pltpu.VMEM_SHARED((NPAD, H), _F32),
        ] + [pltpu.SemaphoreType.DMA] * (2 * NB + 1),
    )
    def k(ea_hbm, dst_hbm, aux_out, dst_v, dsc_v, ea_v, rows_v, zero_v, acc,
          *sems):
        semi = sems[0:NB]
        sems_ = sems[NB:2 * NB]
        semz = sems[2 * NB]
        c = lax.axis_index("c")
        s = lax.axis_index("s")
        wid = s * NC + c
        ebase = wid * EPW
        zvec = jnp.zeros((LN,), _F32)
        lane = lax.iota(_I32, LN)
        onehot = jnp.where(lane == 0, jnp.float32(1.0), jnp.float32(0.0))

        def init_zero(i, carry):
            for j in range(H // LN):
                zero_v[i, pl.ds(j * LN, LN)] = zvec
            return carry

        lax.fori_loop(0, ZR, init_zero, 0)

        def init_rows(i, carry):
            for u in range(NB):
                rows_v[u, i, pl.ds(LN, LN)] = onehot
                for j in range(2, H // LN):
                    rows_v[u, i, pl.ds(j * LN, LN)] = zvec
            return carry

        lax.fori_loop(0, CH, init_rows, 0)
        rbase = s * RPS

        def fetch(i, u):
            b = ebase + i * CH
            pltpu.async_copy(dst_hbm.at[pl.ds(b, CH)], dst_v.at[u], semi[u])
            pltpu.async_copy(ea_hbm.at[pl.ds(b, CH)], ea_v.at[u], semi[u])

        def wait_fetch(u):
            pltpu.make_async_copy(dst_hbm.at[pl.ds(0, CH)], dst_v.at[u],
                                  semi[u]).wait()
            pltpu.make_async_copy(ea_hbm.at[pl.ds(0, CH)], ea_v.at[u],
                                  semi[u]).wait()

        def wait_scat(u):
            pltpu.make_async_copy(z_dummy.at[pl.ds(0, CH)], rows_v.at[u],
                                  sems_[u]).wait()

        z_dummy = aux_out.at[0]

        for u in range(NB):
            fetch(u, u)

        def zacc(i, carry):
            pltpu.async_copy(zero_v, acc.at[pl.ds(rbase + i * ZR, ZR)], semz)
            return carry

        lax.fori_loop(0, RPS // ZR, zacc, 0)

        def zdrain(i, carry):
            pltpu.make_async_copy(z_dummy.at[pl.ds(0, ZR)], zero_v,
                                  semz).wait()
            return carry

        lax.fori_loop(0, RPS // ZR, zdrain, 0)
        plsc.subcore_barrier()

        def stage(i, u):
            @pl.when(i > 0)
            def _():
                wait_scat(u)

            wait_fetch(u)

            def repack(r, carry2):
                rows_v[u, r, pl.ds(0, LN)] = ea_v[u, r, pl.ds(0, LN)]
                return carry2

            lax.fori_loop(0, CH, repack, 0)
            for g in range(CH // LN):
                dsc_v[u, pl.ds(g * LN, LN)] = dst_v[u, pl.ds(g * LN, LN)]
            pltpu.async_copy(rows_v.at[u], acc.at[dsc_v.at[u]], sems_[u],
                             add=True)

        def body(i, carry):
            for u in range(NB):
                idx = i * NB + u
                stage(i, u)

                @pl.when(idx + NB < NIT)
                def _():
                    fetch(idx + NB, u)

            return carry

        lax.fori_loop(0, NIT // NB, body, 0)
        for u in range(NIT - (NIT // NB) * NB):
            stage(NIT // NB, u)
        for u in range(min(NB, NIT)):
            wait_scat(u)
        plsc.subcore_barrier()

        @pl.when(s < NS - 1)
        def _full():
            pltpu.sync_copy(acc.at[pl.ds(rbase, RPS)],
                            aux_out.at[c, pl.ds(rbase, RPS)])

        @pl.when(s == NS - 1)
        def _tail():
            pltpu.sync_copy(acc.at[pl.ds((NS - 1) * RPS, LASTR)],
                            aux_out.at[c, pl.ds((NS - 1) * RPS, LASTR)])

    return k(edge_attr, dst_p)


def _dotT(a, w):
    return lax.dot_general(a, w, (((1,), (1,)), ((), ())),
                           preferred_element_type=_F32)


def _tc_auxred(aux):
    NC, N, H = aux.shape

    def body(a_ref, o_ref):
        o_ref[...] = a_ref[0, :, 0:32] + a_ref[1, :, 0:32]

    return pl.pallas_call(
        body, out_shape=jax.ShapeDtypeStruct((N, 32), _F32))(aux)


def _tc_prep(batch2d, x, nW0, vn_emb, Bsz):
    _, N = batch2d.shape
    H = nW0.shape[0]

    def body(batch_ref, x_ref, w_ref, vne_ref, P_ref, Pn_ref, z_ref, vn_ref):
        iota = lax.broadcasted_iota(_I32, (Bsz, N), 0)
        P = (iota == batch_ref[...]).astype(_F32)
        counts = jnp.sum(P, axis=1, keepdims=True)
        Pn = P / jnp.maximum(counts, 1.0)
        P_ref[...] = P
        Pn_ref[...] = Pn
        vn0 = jnp.broadcast_to(vne_ref[...], (Bsz, H))
        vn_ref[...] = vn0
        z_ref[...] = _dotT(x_ref[...], w_ref[...]) + lax.dot_general(
            P, vn0, (((0,), (0,)), ((), ())), preferred_element_type=_F32)

    return pl.pallas_call(
        body,
        out_shape=(
            jax.ShapeDtypeStruct((Bsz, N), _F32),
            jax.ShapeDtypeStruct((Bsz, N), _F32),
            jax.ShapeDtypeStruct((N, H), _F32),
            jax.ShapeDtypeStruct((Bsz, H), _F32),
        ),
    )(batch2d, x, nW0, vn_emb)


def _tc_layer(Ap, ead, vn, P, Pn, eW, beff, W1, b1, W2, b2,
              vW1, vb1, vW2, vb2, nW_next):
    NC, N, H = Ap.shape
    Bsz = P.shape[0]

    def body(Ap_ref, ead_ref, vn_ref, P_ref, Pn_ref, eW_ref, beff_ref,
             W1_ref, b1_ref, W2_ref, b2_ref, vW1_ref, vb1_ref, vW2_ref,
             vb2_ref, nWn_ref, z_ref, vno_ref):
        A = Ap_ref[0] + Ap_ref[1]
        ead = ead_ref[...]
        EA = ead[:, 0:16]
        deg = ead[:, 16:17]
        agg = A + _dotT(EA, eW_ref[...]) + deg * beff_ref[...]
        t = jnp.maximum(_dotT(agg, W1_ref[...]) + b1_ref[...], 0.0)
        h = jnp.maximum(_dotT(t, W2_ref[...]) + b2_ref[...], 0.0)
        pooled = jnp.dot(Pn_ref[...], h, preferred_element_type=_F32)
        q = jnp.maximum(_dotT(pooled, vW1_ref[...]) + vb1_ref[...], 0.0)
        vnu = jnp.maximum(_dotT(q, vW2_ref[...]) + vb2_ref[...], 0.0)
        vn_new = vn_ref[...] + vnu
        vno_ref[...] = vn_new
        z_ref[...] = _dotT(h, nWn_ref[...]) + lax.dot_general(
            P_ref[...], vn_new, (((0,), (0,)), ((), ())),
            preferred_element_type=_F32)

    return pl.pallas_call(
        body,
        out_shape=(
            jax.ShapeDtypeStruct((N, H), _F32),
            jax.ShapeDtypeStruct((Bsz, H), _F32),
        ),
    )(Ap, ead, vn, P, Pn, eW, beff, W1, b1, W2, b2, vW1, vb1, vW2, vb2,
      nW_next)


def _tc_last(Ap, ead, Pn, eW, beff, W1, b1, W2, b2, fcW, fcb):
    NC, N, H = Ap.shape
    Bsz = Pn.shape[0]
    OUT = fcW.shape[0]

    def body(Ap_ref, ead_ref, Pn_ref, eW_ref, beff_ref, W1_ref,
             b1_ref, W2_ref, b2_ref, fcW_ref, fcb_ref, o_ref):
        A = Ap_ref[0] + Ap_ref[1]
        ead = ead_ref[...]
        EA = ead[:, 0:16]
        deg = ead[:, 16:17]
        agg = A + _dotT(EA, eW_ref[...]) + deg * beff_ref[...]
        t = jnp.maximum(_dotT(agg, W1_ref[...]) + b1_ref[...], 0.0)
        h = jnp.maximum(_dotT(t, W2_ref[...]) + b2_ref[...], 0.0)
        pooled = jnp.dot(Pn_ref[...], h, preferred_element_type=_F32)
        o_ref[...] = _dotT(pooled, fcW_ref[...]) + fcb_ref[...]

    return pl.pallas_call(
        body,
        out_shape=jax.ShapeDtypeStruct((Bsz, OUT), _F32),
    )(Ap, ead, Pn, eW, beff, W1, b1, W2, b2, fcW, fcb)


def kernel(x, edge_index, edge_attr, batch, node_W, node_b, edge_W, edge_b,
           mlp_W1, mlp_b1, mlp_W2, mlp_b2, vn_emb, vnmlp_W1, vnmlp_b1,
           vnmlp_W2, vnmlp_b2, fc_W, fc_b):
    N, F = x.shape
    L, H, _ = node_W.shape
    E = edge_index.shape[1]
    ED = edge_attr.shape[1]
    Bsz = 128
    NC, NS, _LN = _sc_geometry()
    NW = NC * NS
    NPAD, _RPS, _LASTR = _row_partition(N, NS)
    CH = 64
    EPW = -(-E // (NW * CH)) * CH
    pad = NW * EPW - E
    src = edge_index[0]
    dst = edge_index[1]
    src_p = jnp.concatenate([src, jnp.zeros((pad,), _I32)])
    dst_p = jnp.concatenate([dst, jnp.full((pad,), N, _I32)])
    ea_p = jnp.concatenate([edge_attr, jnp.zeros((pad, ED), _F32)])
    r2 = lambda v: v.reshape(1, -1)

    aux = _sc_edgeprep_impl(ea_p, dst_p, N, H, EPW, CH)
    ead = _tc_auxred(aux)
    P, Pn, z, vn = _tc_prep(batch.reshape(1, N), x, node_W[0], vn_emb, Bsz)

    out = None
    for l in range(L):
        Ap = _sc_segsum(z, src_p, dst_p, EPW, CH)
        beff = r2(node_b[l] + edge_b[l])
        if l + 1 < L:
            z, vn = _tc_layer(Ap, ead, vn, P, Pn, edge_W[l], beff,
                              mlp_W1[l], r2(mlp_b1[l]), mlp_W2[l],
                              r2(mlp_b2[l]), vnmlp_W1, r2(vnmlp_b1),
                              vnmlp_W2, r2(vnmlp_b2), node_W[l + 1])
        else:
            out = _tc_last(Ap, ead, Pn, edge_W[l], beff, mlp_W1[l],
                           r2(mlp_b1[l]), mlp_W2[l], r2(mlp_b2[l]), fc_W,
                           r2(fc_b))
    return out


# speedup vs baseline: 1.0355x; 1.0185x over previous
import functools

import jax
import jax.numpy as jnp
from jax import lax
from jax.experimental import pallas as pl
from jax.experimental.pallas import tpu as pltpu
from jax.experimental.pallas import tpu_sc as plsc

_F32 = jnp.float32
_I32 = jnp.int32


def _sc_geometry():
    try:
        info = plsc.get_sparse_core_info()
        return info.num_cores, info.num_subcores, info.num_lanes
    except Exception:
        return 2, 16, 16


def _pick_chunk(epw, cap=128):
    for c in range(cap, 7, -8):
        if epw % c == 0:
            return c
    return 8


def _row_partition(N, NS):
    npad = -(-N // (NS * 8)) * (NS * 8)
    rps = npad // NS
    last = N - (NS - 1) * rps
    return npad, rps, last


def _sc_segsum(z, src_p, dst_p, EPW, CH):
    N, H = z.shape
    NC, NS, LN = _sc_geometry()
    NIT = EPW // CH
    NPAD, RPS, LASTR = _row_partition(N, NS)
    ZR = 8
    mesh = plsc.VectorSubcoreMesh(core_axis_name="c", subcore_axis_name="s")

    NB = 4

    @functools.partial(
        pl.kernel,
        mesh=mesh,
        out_type=jax.ShapeDtypeStruct((NC, N, H), _F32),
        scratch_types=[
            pltpu.VMEM((NB, CH), _I32),
            pltpu.VMEM((NB, CH), _I32),
            pltpu.VMEM((NB, CH, H), _F32),
            pltpu.VMEM((ZR, H), _F32),
            pltpu.VMEM_SHARED((NPAD, H), _F32),
        ] + [pltpu.SemaphoreType.DMA] * (3 * NB + 1),
    )
    def k(z_hbm, src_hbm, dst_hbm, out_hbm, srcb, dstb, rows, zero_v, acc,
          *sems):
        semi = sems[0:NB]
        semg = sems[NB:2 * NB]
        sems_ = sems[2 * NB:3 * NB]
        semz = sems[3 * NB]
        c = lax.axis_index("c")
        s = lax.axis_index("s")
        wid = s * NC + c
        ebase = wid * EPW
        zvec = jnp.zeros((LN,), _F32)

        def zrow(i, carry):
            for j in range(H // LN):
                zero_v[i, pl.ds(j * LN, LN)] = zvec
            return carry

        lax.fori_loop(0, ZR, zrow, 0)
        rbase = s * RPS

        def fetch_idx(i, u):
            pltpu.async_copy(src_hbm.at[pl.ds(ebase + i * CH, CH)],
                             srcb.at[u], semi[u])
            pltpu.async_copy(dst_hbm.at[pl.ds(ebase + i * CH, CH)],
                             dstb.at[u], semi[u])

        def gather(u):
            pltpu.make_async_copy(src_hbm.at[pl.ds(0, CH)], srcb.at[u],
                                  semi[u]).wait()
            pltpu.make_async_copy(dst_hbm.at[pl.ds(0, CH)], dstb.at[u],
                                  semi[u]).wait()
            pltpu.async_copy(z_hbm.at[srcb.at[u]], rows.at[u], semg[u])

        def scatter(u):
            pltpu.make_async_copy(z_hbm.at[pl.ds(0, CH)], rows.at[u],
                                  semg[u]).wait()
            pltpu.async_copy(rows.at[u], acc.at[dstb.at[u]], sems_[u],
                             add=True)

        def wait_scat(u):
            pltpu.make_async_copy(z_hbm.at[pl.ds(0, CH)], rows.at[u],
                                  sems_[u]).wait()

        for u in range(NB):
            if u < NIT:
                fetch_idx(u, u)
        for u in range(NB):
            if u < NIT:
                gather(u)

        def zacc(i, carry):
            pltpu.async_copy(zero_v, acc.at[pl.ds(rbase + i * ZR, ZR)], semz)
            return carry

        lax.fori_loop(0, RPS // ZR, zacc, 0)

        def zdrain(i, carry):
            pltpu.make_async_copy(z_hbm.at[pl.ds(0, ZR)], zero_v, semz).wait()
            return carry

        lax.fori_loop(0, RPS // ZR, zdrain, 0)
        plsc.subcore_barrier()

        NFULL = NIT // NB

        def body(j, carry):
            base = j * NB
            for u in range(NB):
                scatter(u)
            for u in range(NB):
                nxt = base + NB + u

                @pl.when(nxt < NIT)
                def _():
                    wait_scat(u)
                    fetch_idx(nxt, u)
                    gather(u)

            return carry

        lax.fori_loop(0, NFULL, body, 0)
        for u in range(NIT - NFULL * NB):
            scatter(u)
        for u in range(min(NB, NIT)):
            wait_scat(u)
        plsc.subcore_barrier()

        @pl.when(s < NS - 1)
        def _full():
            pltpu.sync_copy(acc.at[pl.ds(rbase, RPS)],
                            out_hbm.at[c, pl.ds(rbase, RPS)])

        @pl.when(s == NS - 1)
        def _tail():
            pltpu.sync_copy(acc.at[pl.ds((NS - 1) * RPS, LASTR)],
                            out_hbm.at[c, pl.ds((NS - 1) * RPS, LASTR)])

    return k(z, src_p, dst_p)


def _sc_edgeprep_impl(edge_attr, dst_p, N, H, CH):
    E, ED = edge_attr.shape
    NC, NS, LN = _sc_geometry()
    EPW = E // (NC * NS)
    NIT = EPW // CH
    CHT = EPW - NIT * CH
    NPAD, RPS, LASTR = _row_partition(N, NS)
    ZR = 8
    mesh = plsc.VectorSubcoreMesh(core_axis_name="c", subcore_axis_name="s")

    NB = 2

    @functools.partial(
        pl.kernel,
        mesh=mesh,
        out_type=jax.ShapeDtypeStruct((NC, N, H), _F32),
        scratch_types=[
            pltpu.VMEM((NB, CH), _I32),
            pltpu.VMEM((NB, CH), _I32),
            pltpu.VMEM((CHT if CHT else 8,), _I32),
            pltpu.VMEM((NB, CH, ED), _F32),
            pltpu.VMEM((NB, CH, H), _F32),
            pltpu.VMEM((ZR, H), _F32),
            pltpu.VMEM_SHARED((NPAD, H), _F32),
        ] + [pltpu.SemaphoreType.DMA] * (2 * NB + 1),
    )
    def k(ea_hbm, dst_hbm, aux_out, dst_v, dsc_v, dstt_v, ea_v, rows_v,
          zero_v, acc, *sems):
        semi = sems[0:NB]
        sems_ = sems[NB:2 * NB]
        semz = sems[2 * NB]
        c = lax.axis_index("c")
        s = lax.axis_index("s")
        wid = s * NC + c
        ebase = wid * EPW
        zvec = jnp.zeros((LN,), _F32)
        lane = lax.iota(_I32, LN)
        onehot = jnp.where(lane == 0, jnp.float32(1.0), jnp.float32(0.0))

        def init_zero(i, carry):
            for j in range(H // LN):
                zero_v[i, pl.ds(j * LN, LN)] = zvec
            return carry

        lax.fori_loop(0, ZR, init_zero, 0)

        def init_rows(i, carry):
            for u in range(NB):
                rows_v[u, i, pl.ds(LN, LN)] = onehot
                for j in range(2, H // LN):
                    rows_v[u, i, pl.ds(j * LN, LN)] = zvec
            return carry

        lax.fori_loop(0, CH, init_rows, 0)
        rbase = s * RPS

        def fetch(i, u):
            b = ebase + i * CH
            pltpu.async_copy(dst_hbm.at[pl.ds(b, CH)], dst_v.at[u], semi[u])
            pltpu.async_copy(ea_hbm.at[pl.ds(b, CH)], ea_v.at[u], semi[u])

        def wait_fetch(u):
            pltpu.make_async_copy(dst_hbm.at[pl.ds(0, CH)], dst_v.at[u],
                                  semi[u]).wait()
            pltpu.make_async_copy(ea_hbm.at[pl.ds(0, CH)], ea_v.at[u],
                                  semi[u]).wait()

        def wait_scat(u):
            pltpu.make_async_copy(z_dummy.at[pl.ds(0, CH)], rows_v.at[u],
                                  sems_[u]).wait()

        z_dummy = aux_out.at[0]

        for u in range(NB):
            fetch(u, u)

        def zacc(i, carry):
            pltpu.async_copy(zero_v, acc.at[pl.ds(rbase + i * ZR, ZR)], semz)
            return carry

        lax.fori_loop(0, RPS // ZR, zacc, 0)

        def zdrain(i, carry):
            pltpu.make_async_copy(z_dummy.at[pl.ds(0, ZR)], zero_v,
                                  semz).wait()
            return carry

        lax.fori_loop(0, RPS // ZR, zdrain, 0)
        plsc.subcore_barrier()

        def stage(i, u):
            @pl.when(i > 0)
            def _():
                wait_scat(u)

            wait_fetch(u)

            def repack(r, carry2):
                rows_v[u, r, pl.ds(0, LN)] = ea_v[u, r, pl.ds(0, LN)]
                return carry2

            lax.fori_loop(0, CH, repack, 0)
            for g in range(CH // LN):
                dsc_v[u, pl.ds(g * LN, LN)] = dst_v[u, pl.ds(g * LN, LN)]
            pltpu.async_copy(rows_v.at[u], acc.at[dsc_v.at[u]], sems_[u],
                             add=True)

        def body(i, carry):
            for u in range(NB):
                idx = i * NB + u
                stage(i, u)

                @pl.when(idx + NB < NIT)
                def _():
                    fetch(idx + NB, u)

            return carry

        lax.fori_loop(0, NIT // NB, body, 0)
        for u in range(NIT - (NIT // NB) * NB):
            stage(NIT // NB, u)
        for u in range(min(NB, NIT)):
            wait_scat(u)
        if CHT:
            b = ebase + NIT * CH
            pltpu.sync_copy(dst_hbm.at[pl.ds(b, CHT)], dstt_v)
            pltpu.sync_copy(ea_hbm.at[pl.ds(b, CHT)], ea_v.at[0, pl.ds(0, CHT)])

            def repack_tail(r, carry2):
                rows_v[0, r, pl.ds(0, LN)] = ea_v[0, r, pl.ds(0, LN)]
                return carry2

            lax.fori_loop(0, CHT, repack_tail, 0)
            pltpu.sync_copy(rows_v.at[0, pl.ds(0, CHT)],
                            acc.at[dstt_v], add=True)
        plsc.subcore_barrier()

        @pl.when(s < NS - 1)
        def _full():
            pltpu.sync_copy(acc.at[pl.ds(rbase, RPS)],
                            aux_out.at[c, pl.ds(rbase, RPS)])

        @pl.when(s == NS - 1)
        def _tail():
            pltpu.sync_copy(acc.at[pl.ds((NS - 1) * RPS, LASTR)],
                            aux_out.at[c, pl.ds((NS - 1) * RPS, LASTR)])

    return k(edge_attr, dst_p)


def _dotT(a, w):
    return lax.dot_general(a, w, (((1,), (1,)), ((), ())),
                           preferred_element_type=_F32)


def _tc_auxred(aux):
    NC, N, H = aux.shape

    def body(a_ref, o_ref):
        o_ref[...] = a_ref[0, :, 0:32] + a_ref[1, :, 0:32]

    return pl.pallas_call(
        body, out_shape=jax.ShapeDtypeStruct((N, 32), _F32))(aux)


def _tc_prep(batch2d, x, nW0, vn_emb, Bsz):
    _, N = batch2d.shape
    H = nW0.shape[0]

    def body(batch_ref, x_ref, w_ref, vne_ref, Pn_ref, cnt_ref, z_ref,
             vn_ref):
        iota = lax.broadcasted_iota(_I32, (Bsz, N), 0)
        P = (iota == batch_ref[...]).astype(_F32)
        counts = jnp.sum(P, axis=1, keepdims=True)
        Pn = P / jnp.maximum(counts, 1.0)
        Pn_ref[...] = Pn
        cnt_ref[...] = counts
        vn0 = jnp.broadcast_to(vne_ref[...], (Bsz, H))
        vn_ref[...] = vn0
        z_ref[...] = _dotT(x_ref[...], w_ref[...]) + lax.dot_general(
            Pn, counts * vn0, (((0,), (0,)), ((), ())),
            preferred_element_type=_F32)

    return pl.pallas_call(
        body,
        out_shape=(
            jax.ShapeDtypeStruct((Bsz, N), _F32),
            jax.ShapeDtypeStruct((Bsz, 1), _F32),
            jax.ShapeDtypeStruct((N, H), _F32),
            jax.ShapeDtypeStruct((Bsz, H), _F32),
        ),
    )(batch2d, x, nW0, vn_emb)


def _tc_layer(Ap, ead, vn, Pn, cnt, eW, beff, W1, b1, W2, b2,
              vW1, vb1, vW2, vb2, nW_next):
    NC, N, H = Ap.shape
    Bsz = Pn.shape[0]

    def body(Ap_ref, ead_ref, vn_ref, Pn_ref, cnt_ref, eW_ref, beff_ref,
             W1_ref, b1_ref, W2_ref, b2_ref, vW1_ref, vb1_ref, vW2_ref,
             vb2_ref, nWn_ref, z_ref, vno_ref):
        A = Ap_ref[0] + Ap_ref[1]
        ead = ead_ref[...]
        EA = ead[:, 0:16]
        deg = ead[:, 16:17]
        agg = A + _dotT(EA, eW_ref[...]) + deg * beff_ref[...]
        t = jnp.maximum(_dotT(agg, W1_ref[...]) + b1_ref[...], 0.0)
        h = jnp.maximum(_dotT(t, W2_ref[...]) + b2_ref[...], 0.0)
        pooled = jnp.dot(Pn_ref[...], h, preferred_element_type=_F32)
        q = jnp.maximum(_dotT(pooled, vW1_ref[...]) + vb1_ref[...], 0.0)
        vnu = jnp.maximum(_dotT(q, vW2_ref[...]) + vb2_ref[...], 0.0)
        vn_new = vn_ref[...] + vnu
        vno_ref[...] = vn_new
        z_ref[...] = _dotT(h, nWn_ref[...]) + lax.dot_general(
            Pn_ref[...], cnt_ref[...] * vn_new, (((0,), (0,)), ((), ())),
            preferred_element_type=_F32)

    return pl.pallas_call(
        body,
        out_shape=(
            jax.ShapeDtypeStruct((N, H), _F32),
            jax.ShapeDtypeStruct((Bsz, H), _F32),
        ),
    )(Ap, ead, vn, Pn, cnt, eW, beff, W1, b1, W2, b2, vW1, vb1, vW2, vb2,
      nW_next)


def _tc_last(Ap, ead, Pn, eW, beff, W1, b1, W2, b2, fcW, fcb):
    NC, N, H = Ap.shape
    Bsz = Pn.shape[0]
    OUT = fcW.shape[0]

    def body(Ap_ref, ead_ref, Pn_ref, eW_ref, beff_ref, W1_ref,
             b1_ref, W2_ref, b2_ref, fcW_ref, fcb_ref, o_ref):
        A = Ap_ref[0] + Ap_ref[1]
        ead = ead_ref[...]
        EA = ead[:, 0:16]
        deg = ead[:, 16:17]
        agg = A + _dotT(EA, eW_ref[...]) + deg * beff_ref[...]
        t = jnp.maximum(_dotT(agg, W1_ref[...]) + b1_ref[...], 0.0)
        h = jnp.maximum(_dotT(t, W2_ref[...]) + b2_ref[...], 0.0)
        pooled = jnp.dot(Pn_ref[...], h, preferred_element_type=_F32)
        o_ref[...] = _dotT(pooled, fcW_ref[...]) + fcb_ref[...]

    return pl.pallas_call(
        body,
        out_shape=jax.ShapeDtypeStruct((Bsz, OUT), _F32),
    )(Ap, ead, Pn, eW, beff, W1, b1, W2, b2, fcW, fcb)


def kernel(x, edge_index, edge_attr, batch, node_W, node_b, edge_W, edge_b,
           mlp_W1, mlp_b1, mlp_W2, mlp_b2, vn_emb, vnmlp_W1, vnmlp_b1,
           vnmlp_W2, vnmlp_b2, fc_W, fc_b):
    N, F = x.shape
    L, H, _ = node_W.shape
    E = edge_index.shape[1]
    ED = edge_attr.shape[1]
    Bsz = 128
    NC, NS, _LN = _sc_geometry()
    NW = NC * NS
    NPAD, _RPS, _LASTR = _row_partition(N, NS)
    CH = 64
    EPW = -(-E // (NW * CH)) * CH
    pad = NW * EPW - E
    src = edge_index[0]
    dst = edge_index[1]
    src_p = jnp.concatenate([src, jnp.zeros((pad,), _I32)])
    dst_p = jnp.concatenate([dst, jnp.full((pad,), N, _I32)])
    r2 = lambda v: v.reshape(1, -1)

    aux = _sc_edgeprep_impl(edge_attr, dst_p, N, H, CH)
    ead = _tc_auxred(aux)
    Pn, cnt, z, vn = _tc_prep(batch.reshape(1, N), x, node_W[0], vn_emb, Bsz)

    out = None
    for l in range(L):
        Ap = _sc_segsum(z, src_p, dst_p, EPW, CH)
        beff = r2(node_b[l] + edge_b[l])
        if l + 1 < L:
            z, vn = _tc_layer(Ap, ead, vn, Pn, cnt, edge_W[l], beff,
                              mlp_W1[l], r2(mlp_b1[l]), mlp_W2[l],
                              r2(mlp_b2[l]), vnmlp_W1, r2(vnmlp_b1),
                              vnmlp_W2, r2(vnmlp_b2), node_W[l + 1])
        else:
            out = _tc_last(Ap, ead, Pn, edge_W[l], beff, mlp_W1[l],
                           r2(mlp_b1[l]), mlp_W2[l], r2(mlp_b2[l]), fc_W,
                           r2(fc_b))
    return out


# speedup vs baseline: 1.3928x; 1.3451x over previous
import functools

import jax
import jax.numpy as jnp
from jax import lax
from jax.experimental import pallas as pl
from jax.experimental.pallas import tpu as pltpu
from jax.experimental.pallas import tpu_sc as plsc

_F32 = jnp.float32
_I32 = jnp.int32


def _sc_geometry():
    try:
        info = plsc.get_sparse_core_info()
        return info.num_cores, info.num_subcores, info.num_lanes
    except Exception:
        return 2, 16, 16


def _pick_chunk(epw, cap=128):
    for c in range(cap, 7, -8):
        if epw % c == 0:
            return c
    return 8


def _row_partition(N, NS):
    npad = -(-N // (NS * 8)) * (NS * 8)
    rps = npad // NS
    last = N - (NS - 1) * rps
    return npad, rps, last


def _sc_segsum(z, src, dst, CH):
    N, H = z.shape
    NC, NS, LN = _sc_geometry()
    EPW = src.shape[0] // (NC * NS)
    NIT = EPW // CH
    CHT = EPW - NIT * CH
    NPAD, RPS, LASTR = _row_partition(N, NS)
    ZR = 8
    mesh = plsc.VectorSubcoreMesh(core_axis_name="c", subcore_axis_name="s")

    NB = 4

    @functools.partial(
        pl.kernel,
        mesh=mesh,
        out_type=jax.ShapeDtypeStruct((NC, N, H), _F32),
        scratch_types=[
            pltpu.VMEM((NB, CH), _I32),
            pltpu.VMEM((NB, CH), _I32),
            pltpu.VMEM((CHT if CHT else 8,), _I32),
            pltpu.VMEM((CHT if CHT else 8,), _I32),
            pltpu.VMEM((NB, CH, H), _F32),
            pltpu.VMEM((ZR, H), _F32),
            pltpu.VMEM_SHARED((NPAD, H), _F32),
        ] + [pltpu.SemaphoreType.DMA] * (3 * NB + 1),
    )
    def k(z_hbm, src_hbm, dst_hbm, out_hbm, srcb, dstb, srct_v, dstt_v,
          rows, zero_v, acc, *sems):
        semi = sems[0:NB]
        semg = sems[NB:2 * NB]
        sems_ = sems[2 * NB:3 * NB]
        semz = sems[3 * NB]
        c = lax.axis_index("c")
        s = lax.axis_index("s")
        wid = s * NC + c
        ebase = wid * EPW
        zvec = jnp.zeros((LN,), _F32)

        def zrow(i, carry):
            for j in range(H // LN):
                zero_v[i, pl.ds(j * LN, LN)] = zvec
            return carry

        lax.fori_loop(0, ZR, zrow, 0)
        rbase = s * RPS

        def fetch_idx(i, u):
            pltpu.async_copy(src_hbm.at[pl.ds(ebase + i * CH, CH)],
                             srcb.at[u], semi[u])
            pltpu.async_copy(dst_hbm.at[pl.ds(ebase + i * CH, CH)],
                             dstb.at[u], semi[u])

        def gather(u):
            pltpu.make_async_copy(src_hbm.at[pl.ds(0, CH)], srcb.at[u],
                                  semi[u]).wait()
            pltpu.make_async_copy(dst_hbm.at[pl.ds(0, CH)], dstb.at[u],
                                  semi[u]).wait()
            pltpu.async_copy(z_hbm.at[srcb.at[u]], rows.at[u], semg[u])

        def scatter(u):
            pltpu.make_async_copy(z_hbm.at[pl.ds(0, CH)], rows.at[u],
                                  semg[u]).wait()
            pltpu.async_copy(rows.at[u], acc.at[dstb.at[u]], sems_[u],
                             add=True)

        def wait_scat(u):
            pltpu.make_async_copy(z_hbm.at[pl.ds(0, CH)], rows.at[u],
                                  sems_[u]).wait()

        for u in range(NB):
            if u < NIT:
                fetch_idx(u, u)
        for u in range(NB):
            if u < NIT:
                gather(u)

        def zacc(i, carry):
            pltpu.async_copy(zero_v, acc.at[pl.ds(rbase + i * ZR, ZR)], semz)
            return carry

        lax.fori_loop(0, RPS // ZR, zacc, 0)

        def zdrain(i, carry):
            pltpu.make_async_copy(z_hbm.at[pl.ds(0, ZR)], zero_v, semz).wait()
            return carry

        lax.fori_loop(0, RPS // ZR, zdrain, 0)
        plsc.subcore_barrier()

        NFULL = NIT // NB

        def body(j, carry):
            base = j * NB
            for u in range(NB):
                scatter(u)
            for u in range(NB):
                nxt = base + NB + u

                @pl.when(nxt < NIT)
                def _():
                    wait_scat(u)
                    fetch_idx(nxt, u)
                    gather(u)

            return carry

        lax.fori_loop(0, NFULL, body, 0)
        for u in range(NIT - NFULL * NB):
            scatter(u)
        for u in range(min(NB, NIT)):
            wait_scat(u)
        if CHT:
            b = ebase + NIT * CH
            pltpu.sync_copy(src_hbm.at[pl.ds(b, CHT)], srct_v)
            pltpu.sync_copy(dst_hbm.at[pl.ds(b, CHT)], dstt_v)
            pltpu.async_copy(z_hbm.at[srct_v], rows.at[0, pl.ds(0, CHT)],
                             semg[0])
            pltpu.make_async_copy(z_hbm.at[pl.ds(0, CHT)],
                                  rows.at[0, pl.ds(0, CHT)], semg[0]).wait()
            pltpu.sync_copy(rows.at[0, pl.ds(0, CHT)], acc.at[dstt_v],
                            add=True)
        plsc.subcore_barrier()

        @pl.when(s < NS - 1)
        def _full():
            pltpu.sync_copy(acc.at[pl.ds(rbase, RPS)],
                            out_hbm.at[c, pl.ds(rbase, RPS)])

        @pl.when(s == NS - 1)
        def _tail():
            pltpu.sync_copy(acc.at[pl.ds((NS - 1) * RPS, LASTR)],
                            out_hbm.at[c, pl.ds((NS - 1) * RPS, LASTR)])

    return k(z, src, dst)


def _sc_edgeprep_impl(edge_attr, dst_p, N, H, CH):
    E, ED = edge_attr.shape
    NC, NS, LN = _sc_geometry()
    EPW = E // (NC * NS)
    NIT = EPW // CH
    CHT = EPW - NIT * CH
    NPAD, RPS, LASTR = _row_partition(N, NS)
    ZR = 8
    mesh = plsc.VectorSubcoreMesh(core_axis_name="c", subcore_axis_name="s")

    NB = 2

    @functools.partial(
        pl.kernel,
        mesh=mesh,
        out_type=jax.ShapeDtypeStruct((NC, N, H), _F32),
        scratch_types=[
            pltpu.VMEM((NB, CH), _I32),
            pltpu.VMEM((NB, CH), _I32),
            pltpu.VMEM((CHT if CHT else 8,), _I32),
            pltpu.VMEM((NB, CH, ED), _F32),
            pltpu.VMEM((NB, CH, H), _F32),
            pltpu.VMEM((ZR, H), _F32),
            pltpu.VMEM_SHARED((NPAD, H), _F32),
        ] + [pltpu.SemaphoreType.DMA] * (2 * NB + 1),
    )
    def k(ea_hbm, dst_hbm, aux_out, dst_v, dsc_v, dstt_v, ea_v, rows_v,
          zero_v, acc, *sems):
        semi = sems[0:NB]
        sems_ = sems[NB:2 * NB]
        semz = sems[2 * NB]
        c = lax.axis_index("c")
        s = lax.axis_index("s")
        wid = s * NC + c
        ebase = wid * EPW
        zvec = jnp.zeros((LN,), _F32)
        lane = lax.iota(_I32, LN)
        onehot = jnp.where(lane == 0, jnp.float32(1.0), jnp.float32(0.0))

        def init_zero(i, carry):
            for j in range(H // LN):
                zero_v[i, pl.ds(j * LN, LN)] = zvec
            return carry

        lax.fori_loop(0, ZR, init_zero, 0)

        def init_rows(i, carry):
            for u in range(NB):
                rows_v[u, i, pl.ds(LN, LN)] = onehot
                for j in range(2, H // LN):
                    rows_v[u, i, pl.ds(j * LN, LN)] = zvec
            return carry

        lax.fori_loop(0, CH, init_rows, 0)
        rbase = s * RPS

        def fetch(i, u):
            b = ebase + i * CH
            pltpu.async_copy(dst_hbm.at[pl.ds(b, CH)], dst_v.at[u], semi[u])
            pltpu.async_copy(ea_hbm.at[pl.ds(b, CH)], ea_v.at[u], semi[u])

        def wait_fetch(u):
            pltpu.make_async_copy(dst_hbm.at[pl.ds(0, CH)], dst_v.at[u],
                                  semi[u]).wait()
            pltpu.make_async_copy(ea_hbm.at[pl.ds(0, CH)], ea_v.at[u],
                                  semi[u]).wait()

        def wait_scat(u):
            pltpu.make_async_copy(z_dummy.at[pl.ds(0, CH)], rows_v.at[u],
                                  sems_[u]).wait()

        z_dummy = aux_out.at[0]

        for u in range(NB):
            fetch(u, u)

        def zacc(i, carry):
            pltpu.async_copy(zero_v, acc.at[pl.ds(rbase + i * ZR, ZR)], semz)
            return carry

        lax.fori_loop(0, RPS // ZR, zacc, 0)

        def zdrain(i, carry):
            pltpu.make_async_copy(z_dummy.at[pl.ds(0, ZR)], zero_v,
                                  semz).wait()
            return carry

        lax.fori_loop(0, RPS // ZR, zdrain, 0)
        plsc.subcore_barrier()

        def stage(i, u):
            @pl.when(i > 0)
            def _():
                wait_scat(u)

            wait_fetch(u)

            def repack(r, carry2):
                rows_v[u, r, pl.ds(0, LN)] = ea_v[u, r, pl.ds(0, LN)]
                return carry2

            lax.fori_loop(0, CH, repack, 0)
            for g in range(CH // LN):
                dsc_v[u, pl.ds(g * LN, LN)] = dst_v[u, pl.ds(g * LN, LN)]
            pltpu.async_copy(rows_v.at[u], acc.at[dsc_v.at[u]], sems_[u],
                             add=True)

        def body(i, carry):
            for u in range(NB):
                idx = i * NB + u
                stage(i, u)

                @pl.when(idx + NB < NIT)
                def _():
                    fetch(idx + NB, u)

            return carry

        lax.fori_loop(0, NIT // NB, body, 0)
        for u in range(NIT - (NIT // NB) * NB):
            stage(NIT // NB, u)
        for u in range(min(NB, NIT)):
            wait_scat(u)
        if CHT:
            b = ebase + NIT * CH
            pltpu.sync_copy(dst_hbm.at[pl.ds(b, CHT)], dstt_v)
            pltpu.sync_copy(ea_hbm.at[pl.ds(b, CHT)], ea_v.at[0, pl.ds(0, CHT)])

            def repack_tail(r, carry2):
                rows_v[0, r, pl.ds(0, LN)] = ea_v[0, r, pl.ds(0, LN)]
                return carry2

            lax.fori_loop(0, CHT, repack_tail, 0)
            pltpu.sync_copy(rows_v.at[0, pl.ds(0, CHT)],
                            acc.at[dstt_v], add=True)
        plsc.subcore_barrier()

        @pl.when(s < NS - 1)
        def _full():
            pltpu.sync_copy(acc.at[pl.ds(rbase, RPS)],
                            aux_out.at[c, pl.ds(rbase, RPS)])

        @pl.when(s == NS - 1)
        def _tail():
            pltpu.sync_copy(acc.at[pl.ds((NS - 1) * RPS, LASTR)],
                            aux_out.at[c, pl.ds((NS - 1) * RPS, LASTR)])

    return k(edge_attr, dst_p)


def _dotT(a, w):
    return lax.dot_general(a, w, (((1,), (1,)), ((), ())),
                           preferred_element_type=_F32)


def _tc_auxred(aux):
    NC, N, H = aux.shape

    def body(a_ref, o_ref):
        o_ref[...] = a_ref[0, :, 0:32] + a_ref[1, :, 0:32]

    return pl.pallas_call(
        body, out_shape=jax.ShapeDtypeStruct((N, 32), _F32))(aux)


def _tc_prep(batch2d, x, nW0, vn_emb, Bsz):
    _, N = batch2d.shape
    H = nW0.shape[0]

    def body(batch_ref, x_ref, w_ref, vne_ref, Pn_ref, cnt_ref, z_ref,
             vn_ref):
        iota = lax.broadcasted_iota(_I32, (Bsz, N), 0)
        P = (iota == batch_ref[...]).astype(_F32)
        counts = jnp.sum(P, axis=1, keepdims=True)
        Pn = P / jnp.maximum(counts, 1.0)
        Pn_ref[...] = Pn
        cnt_ref[...] = counts
        vn0 = jnp.broadcast_to(vne_ref[...], (Bsz, H))
        vn_ref[...] = vn0
        z_ref[...] = _dotT(x_ref[...], w_ref[...]) + lax.dot_general(
            Pn, counts * vn0, (((0,), (0,)), ((), ())),
            preferred_element_type=_F32)

    return pl.pallas_call(
        body,
        out_shape=(
            jax.ShapeDtypeStruct((Bsz, N), _F32),
            jax.ShapeDtypeStruct((Bsz, 1), _F32),
            jax.ShapeDtypeStruct((N, H), _F32),
            jax.ShapeDtypeStruct((Bsz, H), _F32),
        ),
    )(batch2d, x, nW0, vn_emb)


def _tc_layer(Ap, ead, vn, Pn, cnt, eW, beff, W1, b1, W2, b2,
              vW1, vb1, vW2, vb2, nW_next):
    NC, N, H = Ap.shape
    Bsz = Pn.shape[0]

    def body(Ap_ref, ead_ref, vn_ref, Pn_ref, cnt_ref, eW_ref, beff_ref,
             W1_ref, b1_ref, W2_ref, b2_ref, vW1_ref, vb1_ref, vW2_ref,
             vb2_ref, nWn_ref, z_ref, vno_ref):
        A = Ap_ref[0] + Ap_ref[1]
        ead = ead_ref[...]
        EA = ead[:, 0:16]
        deg = ead[:, 16:17]
        agg = A + _dotT(EA, eW_ref[...]) + deg * beff_ref[...]
        t = jnp.maximum(_dotT(agg, W1_ref[...]) + b1_ref[...], 0.0)
        h = jnp.maximum(_dotT(t, W2_ref[...]) + b2_ref[...], 0.0)
        pooled = jnp.dot(Pn_ref[...], h, preferred_element_type=_F32)
        q = jnp.maximum(_dotT(pooled, vW1_ref[...]) + vb1_ref[...], 0.0)
        vnu = jnp.maximum(_dotT(q, vW2_ref[...]) + vb2_ref[...], 0.0)
        vn_new = vn_ref[...] + vnu
        vno_ref[...] = vn_new
        z_ref[...] = _dotT(h, nWn_ref[...]) + lax.dot_general(
            Pn_ref[...], cnt_ref[...] * vn_new, (((0,), (0,)), ((), ())),
            preferred_element_type=_F32)

    return pl.pallas_call(
        body,
        out_shape=(
            jax.ShapeDtypeStruct((N, H), _F32),
            jax.ShapeDtypeStruct((Bsz, H), _F32),
        ),
    )(Ap, ead, vn, Pn, cnt, eW, beff, W1, b1, W2, b2, vW1, vb1, vW2, vb2,
      nW_next)


def _tc_last(Ap, ead, Pn, eW, beff, W1, b1, W2, b2, fcW, fcb):
    NC, N, H = Ap.shape
    Bsz = Pn.shape[0]
    OUT = fcW.shape[0]

    def body(Ap_ref, ead_ref, Pn_ref, eW_ref, beff_ref, W1_ref,
             b1_ref, W2_ref, b2_ref, fcW_ref, fcb_ref, o_ref):
        A = Ap_ref[0] + Ap_ref[1]
        ead = ead_ref[...]
        EA = ead[:, 0:16]
        deg = ead[:, 16:17]
        agg = A + _dotT(EA, eW_ref[...]) + deg * beff_ref[...]
        t = jnp.maximum(_dotT(agg, W1_ref[...]) + b1_ref[...], 0.0)
        h = jnp.maximum(_dotT(t, W2_ref[...]) + b2_ref[...], 0.0)
        pooled = jnp.dot(Pn_ref[...], h, preferred_element_type=_F32)
        o_ref[...] = _dotT(pooled, fcW_ref[...]) + fcb_ref[...]

    return pl.pallas_call(
        body,
        out_shape=jax.ShapeDtypeStruct((Bsz, OUT), _F32),
    )(Ap, ead, Pn, eW, beff, W1, b1, W2, b2, fcW, fcb)


def kernel(x, edge_index, edge_attr, batch, node_W, node_b, edge_W, edge_b,
           mlp_W1, mlp_b1, mlp_W2, mlp_b2, vn_emb, vnmlp_W1, vnmlp_b1,
           vnmlp_W2, vnmlp_b2, fc_W, fc_b):
    N, F = x.shape
    L, H, _ = node_W.shape
    Bsz = 128
    CH = 64
    src = edge_index[0]
    dst = edge_index[1]
    r2 = lambda v: v.reshape(1, -1)

    aux = _sc_edgeprep_impl(edge_attr, dst, N, H, CH)
    ead = _tc_auxred(aux)
    Pn, cnt, z, vn = _tc_prep(batch.reshape(1, N), x, node_W[0], vn_emb, Bsz)

    out = None
    for l in range(L):
        Ap = _sc_segsum(z, src, dst, CH)
        beff = r2(node_b[l] + edge_b[l])
        if l + 1 < L:
            z, vn = _tc_layer(Ap, ead, vn, Pn, cnt, edge_W[l], beff,
                              mlp_W1[l], r2(mlp_b1[l]), mlp_W2[l],
                              r2(mlp_b2[l]), vnmlp_W1, r2(vnmlp_b1),
                              vnmlp_W2, r2(vnmlp_b2), node_W[l + 1])
        else:
            out = _tc_last(Ap, ead, Pn, edge_W[l], beff, mlp_W1[l],
                           r2(mlp_b1[l]), mlp_W2[l], r2(mlp_b2[l]), fc_W,
                           r2(fc_b))
    return out


# speedup vs baseline: 1.4640x; 1.0511x over previous
import functools

import jax
import jax.numpy as jnp
from jax import lax
from jax.experimental import pallas as pl
from jax.experimental.pallas import tpu as pltpu
from jax.experimental.pallas import tpu_sc as plsc

_F32 = jnp.float32
_I32 = jnp.int32


def _sc_geometry():
    try:
        info = plsc.get_sparse_core_info()
        return info.num_cores, info.num_subcores, info.num_lanes
    except Exception:
        return 2, 16, 16


def _pick_chunk(epw, cap=128):
    for c in range(cap, 7, -8):
        if epw % c == 0:
            return c
    return 8


def _row_partition(N, NS):
    npad = -(-N // (NS * 8)) * (NS * 8)
    rps = npad // NS
    last = N - (NS - 1) * rps
    return npad, rps, last


def _sc_segsum(z, src, dst, CH):
    N, H = z.shape
    NC, NS, LN = _sc_geometry()
    EPW = src.shape[0] // (NC * NS)
    NIT = EPW // CH
    CHT = EPW - NIT * CH
    NPAD, RPS, LASTR = _row_partition(N, NS)
    ZR = 8
    mesh = plsc.VectorSubcoreMesh(core_axis_name="c", subcore_axis_name="s")

    NB = 5

    @functools.partial(
        pl.kernel,
        mesh=mesh,
        out_type=jax.ShapeDtypeStruct((NC, N, H), _F32),
        scratch_types=[
            pltpu.VMEM((NB, CH), _I32),
            pltpu.VMEM((NB, CH), _I32),
            pltpu.VMEM((CHT if CHT else 8,), _I32),
            pltpu.VMEM((CHT if CHT else 8,), _I32),
            pltpu.VMEM((NB, CH, H), _F32),
            pltpu.VMEM((ZR, H), _F32),
            pltpu.VMEM_SHARED((NPAD, H), _F32),
        ] + [pltpu.SemaphoreType.DMA] * (3 * NB + 1),
    )
    def k(z_hbm, src_hbm, dst_hbm, out_hbm, srcb, dstb, srct_v, dstt_v,
          rows, zero_v, acc, *sems):
        semi = sems[0:NB]
        semg = sems[NB:2 * NB]
        sems_ = sems[2 * NB:3 * NB]
        semz = sems[3 * NB]
        c = lax.axis_index("c")
        s = lax.axis_index("s")
        wid = s * NC + c
        ebase = wid * EPW
        zvec = jnp.zeros((LN,), _F32)

        def zrow(i, carry):
            for j in range(H // LN):
                zero_v[i, pl.ds(j * LN, LN)] = zvec
            return carry

        lax.fori_loop(0, ZR, zrow, 0)
        rbase = s * RPS

        def fetch_idx(i, u):
            pltpu.async_copy(src_hbm.at[pl.ds(ebase + i * CH, CH)],
                             srcb.at[u], semi[u])
            pltpu.async_copy(dst_hbm.at[pl.ds(ebase + i * CH, CH)],
                             dstb.at[u], semi[u])

        def gather(u):
            pltpu.make_async_copy(src_hbm.at[pl.ds(0, CH)], srcb.at[u],
                                  semi[u]).wait()
            pltpu.make_async_copy(dst_hbm.at[pl.ds(0, CH)], dstb.at[u],
                                  semi[u]).wait()
            pltpu.async_copy(z_hbm.at[srcb.at[u]], rows.at[u], semg[u])

        def scatter(u):
            pltpu.make_async_copy(z_hbm.at[pl.ds(0, CH)], rows.at[u],
                                  semg[u]).wait()
            pltpu.async_copy(rows.at[u], acc.at[dstb.at[u]], sems_[u],
                             add=True)

        def wait_scat(u):
            pltpu.make_async_copy(z_hbm.at[pl.ds(0, CH)], rows.at[u],
                                  sems_[u]).wait()

        for u in range(NB):
            if u < NIT:
                fetch_idx(u, u)
        for u in range(NB):
            if u < NIT:
                gather(u)

        def zacc(i, carry):
            pltpu.async_copy(zero_v, acc.at[pl.ds(rbase + i * ZR, ZR)], semz)
            return carry

        lax.fori_loop(0, RPS // ZR, zacc, 0)

        def zdrain(i, carry):
            pltpu.make_async_copy(z_hbm.at[pl.ds(0, ZR)], zero_v, semz).wait()
            return carry

        lax.fori_loop(0, RPS // ZR, zdrain, 0)
        plsc.subcore_barrier()

        NFULL = NIT // NB

        def body(j, carry):
            base = j * NB
            for u in range(NB):
                scatter(u)
            for u in range(NB):
                nxt = base + NB + u

                @pl.when(nxt < NIT)
                def _():
                    wait_scat(u)
                    fetch_idx(nxt, u)
                    gather(u)

            return carry

        lax.fori_loop(0, NFULL, body, 0)
        for u in range(NIT - NFULL * NB):
            scatter(u)
        for u in range(min(NB, NIT)):
            wait_scat(u)
        if CHT:
            b = ebase + NIT * CH
            pltpu.sync_copy(src_hbm.at[pl.ds(b, CHT)], srct_v)
            pltpu.sync_copy(dst_hbm.at[pl.ds(b, CHT)], dstt_v)
            pltpu.async_copy(z_hbm.at[srct_v], rows.at[0, pl.ds(0, CHT)],
                             semg[0])
            pltpu.make_async_copy(z_hbm.at[pl.ds(0, CHT)],
                                  rows.at[0, pl.ds(0, CHT)], semg[0]).wait()
            pltpu.sync_copy(rows.at[0, pl.ds(0, CHT)], acc.at[dstt_v],
                            add=True)
        plsc.subcore_barrier()

        @pl.when(s < NS - 1)
        def _full():
            pltpu.sync_copy(acc.at[pl.ds(rbase, RPS)],
                            out_hbm.at[c, pl.ds(rbase, RPS)])

        @pl.when(s == NS - 1)
        def _tail():
            pltpu.sync_copy(acc.at[pl.ds((NS - 1) * RPS, LASTR)],
                            out_hbm.at[c, pl.ds((NS - 1) * RPS, LASTR)])

    return k(z, src, dst)


def _sc_edgeprep_impl(ea_flat, dst_p, N, H, ED, CH):
    E = ea_flat.shape[0] // ED
    NC, NS, LN = _sc_geometry()
    EPW = E // (NC * NS)
    NIT = EPW // CH
    CHT = EPW - NIT * CH
    NPAD, RPS, LASTR = _row_partition(N, NS)
    ZR = 8
    mesh = plsc.VectorSubcoreMesh(core_axis_name="c", subcore_axis_name="s")

    NB = 2

    @functools.partial(
        pl.kernel,
        mesh=mesh,
        out_type=jax.ShapeDtypeStruct((NC, N, H), _F32),
        scratch_types=[
            pltpu.VMEM((NB, CH), _I32),
            pltpu.VMEM((NB, CH), _I32),
            pltpu.VMEM((CHT if CHT else 8,), _I32),
            pltpu.VMEM((NB, CH * ED), _F32),
            pltpu.VMEM((NB, CH, H), _F32),
            pltpu.VMEM((ZR, H), _F32),
            pltpu.VMEM_SHARED((NPAD, H), _F32),
        ] + [pltpu.SemaphoreType.DMA] * (2 * NB + 1),
    )
    def k(ea_hbm, dst_hbm, aux_out, dst_v, dsc_v, dstt_v, ea_v, rows_v,
          zero_v, acc, *sems):
        semi = sems[0:NB]
        sems_ = sems[NB:2 * NB]
        semz = sems[2 * NB]
        c = lax.axis_index("c")
        s = lax.axis_index("s")
        wid = s * NC + c
        ebase = wid * EPW
        zvec = jnp.zeros((LN,), _F32)
        lane = lax.iota(_I32, LN)
        onehot = jnp.where(lane == 0, jnp.float32(1.0), jnp.float32(0.0))

        def init_zero(i, carry):
            for j in range(H // LN):
                zero_v[i, pl.ds(j * LN, LN)] = zvec
            return carry

        lax.fori_loop(0, ZR, init_zero, 0)

        def init_rows(i, carry):
            for u in range(NB):
                rows_v[u, i, pl.ds(LN, LN)] = onehot
                for j in range(2, H // LN):
                    rows_v[u, i, pl.ds(j * LN, LN)] = zvec
            return carry

        lax.fori_loop(0, CH, init_rows, 0)
        rbase = s * RPS

        def fetch(i, u):
            b = ebase + i * CH
            pltpu.async_copy(dst_hbm.at[pl.ds(b, CH)], dst_v.at[u], semi[u])
            pltpu.async_copy(ea_hbm.at[pl.ds(b * ED, CH * ED)], ea_v.at[u],
                             semi[u])

        def wait_fetch(u):
            pltpu.make_async_copy(dst_hbm.at[pl.ds(0, CH)], dst_v.at[u],
                                  semi[u]).wait()
            pltpu.make_async_copy(ea_hbm.at[pl.ds(0, CH * ED)], ea_v.at[u],
                                  semi[u]).wait()

        def wait_scat(u):
            pltpu.make_async_copy(z_dummy.at[pl.ds(0, CH)], rows_v.at[u],
                                  sems_[u]).wait()

        z_dummy = aux_out.at[0]

        for u in range(NB):
            fetch(u, u)

        def zacc(i, carry):
            pltpu.async_copy(zero_v, acc.at[pl.ds(rbase + i * ZR, ZR)], semz)
            return carry

        lax.fori_loop(0, RPS // ZR, zacc, 0)

        def zdrain(i, carry):
            pltpu.make_async_copy(z_dummy.at[pl.ds(0, ZR)], zero_v,
                                  semz).wait()
            return carry

        lax.fori_loop(0, RPS // ZR, zdrain, 0)
        plsc.subcore_barrier()

        def stage(i, u):
            @pl.when(i > 0)
            def _():
                wait_scat(u)

            wait_fetch(u)

            def repack(r, carry2):
                rows_v[u, r, pl.ds(0, LN)] = ea_v[u, pl.ds(r * ED, LN)]
                return carry2

            lax.fori_loop(0, CH, repack, 0)
            for g in range(CH // LN):
                dsc_v[u, pl.ds(g * LN, LN)] = dst_v[u, pl.ds(g * LN, LN)]
            pltpu.async_copy(rows_v.at[u], acc.at[dsc_v.at[u]], sems_[u],
                             add=True)

        def body(i, carry):
            for u in range(NB):
                idx = i * NB + u
                stage(i, u)

                @pl.when(idx + NB < NIT)
                def _():
                    fetch(idx + NB, u)

            return carry

        lax.fori_loop(0, NIT // NB, body, 0)
        for u in range(NIT - (NIT // NB) * NB):
            stage(NIT // NB, u)
        for u in range(min(NB, NIT)):
            wait_scat(u)
        if CHT:
            b = ebase + NIT * CH
            pltpu.sync_copy(dst_hbm.at[pl.ds(b, CHT)], dstt_v)
            pltpu.sync_copy(ea_hbm.at[pl.ds(b * ED, CHT * ED)],
                            ea_v.at[0, pl.ds(0, CHT * ED)])

            def repack_tail(r, carry2):
                rows_v[0, r, pl.ds(0, LN)] = ea_v[0, pl.ds(r * ED, LN)]
                return carry2

            lax.fori_loop(0, CHT, repack_tail, 0)
            pltpu.sync_copy(rows_v.at[0, pl.ds(0, CHT)],
                            acc.at[dstt_v], add=True)
        plsc.subcore_barrier()

        @pl.when(s < NS - 1)
        def _full():
            pltpu.sync_copy(acc.at[pl.ds(rbase, RPS)],
                            aux_out.at[c, pl.ds(rbase, RPS)])

        @pl.when(s == NS - 1)
        def _tail():
            pltpu.sync_copy(acc.at[pl.ds((NS - 1) * RPS, LASTR)],
                            aux_out.at[c, pl.ds((NS - 1) * RPS, LASTR)])

    return k(ea_flat, dst_p)


def _dotT(a, w):
    return lax.dot_general(a, w, (((1,), (1,)), ((), ())),
                           preferred_element_type=_F32)


def _tc_auxred(aux):
    NC, N, H = aux.shape

    def body(a_ref, o_ref):
        o_ref[...] = a_ref[0, :, 0:32] + a_ref[1, :, 0:32]

    return pl.pallas_call(
        body, out_shape=jax.ShapeDtypeStruct((N, 32), _F32))(aux)


def _tc_prep(batch2d, x, nW0, vn_emb, Bsz):
    _, N = batch2d.shape
    H = nW0.shape[0]

    def body(batch_ref, x_ref, w_ref, vne_ref, Pn_ref, cnt_ref, z_ref,
             vn_ref):
        iota = lax.broadcasted_iota(_I32, (Bsz, N), 0)
        P = (iota == batch_ref[...]).astype(_F32)
        counts = jnp.sum(P, axis=1, keepdims=True)
        Pn = P / jnp.maximum(counts, 1.0)
        Pn_ref[...] = Pn
        cnt_ref[...] = counts
        vn0 = jnp.broadcast_to(vne_ref[...], (Bsz, H))
        vn_ref[...] = vn0
        z_ref[...] = _dotT(x_ref[...], w_ref[...]) + lax.dot_general(
            Pn, counts * vn0, (((0,), (0,)), ((), ())),
            preferred_element_type=_F32)

    return pl.pallas_call(
        body,
        out_shape=(
            jax.ShapeDtypeStruct((Bsz, N), _F32),
            jax.ShapeDtypeStruct((Bsz, 1), _F32),
            jax.ShapeDtypeStruct((N, H), _F32),
            jax.ShapeDtypeStruct((Bsz, H), _F32),
        ),
    )(batch2d, x, nW0, vn_emb)


def _tc_layer(Ap, ead, vn, Pn, cnt, eW, beff, W1, b1, W2, b2,
              vW1, vb1, vW2, vb2, nW_next):
    NC, N, H = Ap.shape
    Bsz = Pn.shape[0]

    def body(Ap_ref, ead_ref, vn_ref, Pn_ref, cnt_ref, eW_ref, beff_ref,
             W1_ref, b1_ref, W2_ref, b2_ref, vW1_ref, vb1_ref, vW2_ref,
             vb2_ref, nWn_ref, z_ref, vno_ref):
        A = Ap_ref[0] + Ap_ref[1]
        ead = ead_ref[...]
        EA = ead[:, 0:16]
        deg = ead[:, 16:17]
        agg = A + _dotT(EA, eW_ref[...]) + deg * beff_ref[...]
        t = jnp.maximum(_dotT(agg, W1_ref[...]) + b1_ref[...], 0.0)
        h = jnp.maximum(_dotT(t, W2_ref[...]) + b2_ref[...], 0.0)
        pooled = jnp.dot(Pn_ref[...], h, preferred_element_type=_F32)
        q = jnp.maximum(_dotT(pooled, vW1_ref[...]) + vb1_ref[...], 0.0)
        vnu = jnp.maximum(_dotT(q, vW2_ref[...]) + vb2_ref[...], 0.0)
        vn_new = vn_ref[...] + vnu
        vno_ref[...] = vn_new
        z_ref[...] = _dotT(h, nWn_ref[...]) + lax.dot_general(
            Pn_ref[...], cnt_ref[...] * vn_new, (((0,), (0,)), ((), ())),
            preferred_element_type=_F32)

    return pl.pallas_call(
        body,
        out_shape=(
            jax.ShapeDtypeStruct((N, H), _F32),
            jax.ShapeDtypeStruct((Bsz, H), _F32),
        ),
    )(Ap, ead, vn, Pn, cnt, eW, beff, W1, b1, W2, b2, vW1, vb1, vW2, vb2,
      nW_next)


def _tc_last(Ap, ead, Pn, eW, beff, W1, b1, W2, b2, fcW, fcb):
    NC, N, H = Ap.shape
    Bsz = Pn.shape[0]
    OUT = fcW.shape[0]

    def body(Ap_ref, ead_ref, Pn_ref, eW_ref, beff_ref, W1_ref,
             b1_ref, W2_ref, b2_ref, fcW_ref, fcb_ref, o_ref):
        A = Ap_ref[0] + Ap_ref[1]
        ead = ead_ref[...]
        EA = ead[:, 0:16]
        deg = ead[:, 16:17]
        agg = A + _dotT(EA, eW_ref[...]) + deg * beff_ref[...]
        t = jnp.maximum(_dotT(agg, W1_ref[...]) + b1_ref[...], 0.0)
        h = jnp.maximum(_dotT(t, W2_ref[...]) + b2_ref[...], 0.0)
        pooled = jnp.dot(Pn_ref[...], h, preferred_element_type=_F32)
        o_ref[...] = _dotT(pooled, fcW_ref[...]) + fcb_ref[...]

    return pl.pallas_call(
        body,
        out_shape=jax.ShapeDtypeStruct((Bsz, OUT), _F32),
    )(Ap, ead, Pn, eW, beff, W1, b1, W2, b2, fcW, fcb)


def kernel(x, edge_index, edge_attr, batch, node_W, node_b, edge_W, edge_b,
           mlp_W1, mlp_b1, mlp_W2, mlp_b2, vn_emb, vnmlp_W1, vnmlp_b1,
           vnmlp_W2, vnmlp_b2, fc_W, fc_b):
    N, F = x.shape
    L, H, _ = node_W.shape
    Bsz = 128
    CH = 64
    src = edge_index[0]
    dst = edge_index[1]
    r2 = lambda v: v.reshape(1, -1)

    aux = _sc_edgeprep_impl(edge_attr.reshape(-1), dst, N, H,
                            edge_attr.shape[1], CH)
    ead = _tc_auxred(aux)
    Pn, cnt, z, vn = _tc_prep(batch.reshape(1, N), x, node_W[0], vn_emb, Bsz)

    out = None
    for l in range(L):
        Ap = _sc_segsum(z, src, dst, CH)
        beff = r2(node_b[l] + edge_b[l])
        if l + 1 < L:
            z, vn = _tc_layer(Ap, ead, vn, Pn, cnt, edge_W[l], beff,
                              mlp_W1[l], r2(mlp_b1[l]), mlp_W2[l],
                              r2(mlp_b2[l]), vnmlp_W1, r2(vnmlp_b1),
                              vnmlp_W2, r2(vnmlp_b2), node_W[l + 1])
        else:
            out = _tc_last(Ap, ead, Pn, edge_W[l], beff, mlp_W1[l],
                           r2(mlp_b1[l]), mlp_W2[l], r2(mlp_b2[l]), fc_W,
                           r2(fc_b))
    return out


# speedup vs baseline: 1.4730x; 1.0061x over previous
import functools

import jax
import jax.numpy as jnp
from jax import lax
from jax.experimental import pallas as pl
from jax.experimental.pallas import tpu as pltpu
from jax.experimental.pallas import tpu_sc as plsc

_F32 = jnp.float32
_I32 = jnp.int32


def _sc_geometry():
    try:
        info = plsc.get_sparse_core_info()
        return info.num_cores, info.num_subcores, info.num_lanes
    except Exception:
        return 2, 16, 16


def _pick_chunk(epw, cap=128):
    for c in range(cap, 7, -8):
        if epw % c == 0:
            return c
    return 8


def _row_partition(N, NS):
    npad = -(-N // (NS * 8)) * (NS * 8)
    rps = npad // NS
    last = N - (NS - 1) * rps
    return npad, rps, last


def _sc_segsum(z, src, dst, CH):
    N, H = z.shape
    NC, NS, LN = _sc_geometry()
    EPW = src.shape[0] // (NC * NS)
    NIT = EPW // CH
    CHT = EPW - NIT * CH
    NPAD, RPS, LASTR = _row_partition(N, NS)
    ZR = 8
    mesh = plsc.VectorSubcoreMesh(core_axis_name="c", subcore_axis_name="s")

    NB = 5

    @functools.partial(
        pl.kernel,
        mesh=mesh,
        out_type=jax.ShapeDtypeStruct((NC, N, H), _F32),
        scratch_types=[
            pltpu.VMEM((NB, CH), _I32),
            pltpu.VMEM((NB, CH), _I32),
            pltpu.VMEM((CHT if CHT else 8,), _I32),
            pltpu.VMEM((CHT if CHT else 8,), _I32),
            pltpu.VMEM((NB, CH, H), _F32),
            pltpu.VMEM((ZR, H), _F32),
            pltpu.VMEM_SHARED((NPAD, H), _F32),
        ] + [pltpu.SemaphoreType.DMA] * (3 * NB + 1),
    )
    def k(z_hbm, src_hbm, dst_hbm, out_hbm, srcb, dstb, srct_v, dstt_v,
          rows, zero_v, acc, *sems):
        semi = sems[0:NB]
        semg = sems[NB:2 * NB]
        sems_ = sems[2 * NB:3 * NB]
        semz = sems[3 * NB]
        c = lax.axis_index("c")
        s = lax.axis_index("s")
        wid = s * NC + c
        ebase = wid * EPW
        zvec = jnp.zeros((LN,), _F32)

        def zrow(i, carry):
            for j in range(H // LN):
                zero_v[i, pl.ds(j * LN, LN)] = zvec
            return carry

        lax.fori_loop(0, ZR, zrow, 0)
        rbase = s * RPS

        def fetch_idx(i, u):
            pltpu.async_copy(src_hbm.at[pl.ds(ebase + i * CH, CH)],
                             srcb.at[u], semi[u])
            pltpu.async_copy(dst_hbm.at[pl.ds(ebase + i * CH, CH)],
                             dstb.at[u], semi[u])

        def gather(u):
            pltpu.make_async_copy(src_hbm.at[pl.ds(0, CH)], srcb.at[u],
                                  semi[u]).wait()
            pltpu.make_async_copy(dst_hbm.at[pl.ds(0, CH)], dstb.at[u],
                                  semi[u]).wait()
            pltpu.async_copy(z_hbm.at[srcb.at[u]], rows.at[u], semg[u])

        def scatter(u):
            pltpu.make_async_copy(z_hbm.at[pl.ds(0, CH)], rows.at[u],
                                  semg[u]).wait()
            pltpu.async_copy(rows.at[u], acc.at[dstb.at[u]], sems_[u],
                             add=True)

        def wait_scat(u):
            pltpu.make_async_copy(z_hbm.at[pl.ds(0, CH)], rows.at[u],
                                  sems_[u]).wait()

        for u in range(NB):
            if u < NIT:
                fetch_idx(u, u)
        for u in range(NB):
            if u < NIT:
                gather(u)

        def zacc(i, carry):
            pltpu.async_copy(zero_v, acc.at[pl.ds(rbase + i * ZR, ZR)], semz)
            return carry

        lax.fori_loop(0, RPS // ZR, zacc, 0)

        def zdrain(i, carry):
            pltpu.make_async_copy(z_hbm.at[pl.ds(0, ZR)], zero_v, semz).wait()
            return carry

        lax.fori_loop(0, RPS // ZR, zdrain, 0)
        plsc.subcore_barrier()

        NFULL = NIT // NB

        def body(j, carry):
            base = j * NB
            for u in range(NB):
                scatter(u)
            for u in range(NB):
                nxt = base + NB + u

                @pl.when(nxt < NIT)
                def _():
                    wait_scat(u)
                    fetch_idx(nxt, u)
                    gather(u)

            return carry

        lax.fori_loop(0, NFULL, body, 0)
        for u in range(NIT - NFULL * NB):
            scatter(u)
        for u in range(min(NB, NIT)):
            wait_scat(u)
        if CHT:
            b = ebase + NIT * CH
            pltpu.sync_copy(src_hbm.at[pl.ds(b, CHT)], srct_v)
            pltpu.sync_copy(dst_hbm.at[pl.ds(b, CHT)], dstt_v)
            pltpu.async_copy(z_hbm.at[srct_v], rows.at[0, pl.ds(0, CHT)],
                             semg[0])
            pltpu.make_async_copy(z_hbm.at[pl.ds(0, CHT)],
                                  rows.at[0, pl.ds(0, CHT)], semg[0]).wait()
            pltpu.sync_copy(rows.at[0, pl.ds(0, CHT)], acc.at[dstt_v],
                            add=True)
        plsc.subcore_barrier()

        @pl.when(s < NS - 1)
        def _full():
            pltpu.sync_copy(acc.at[pl.ds(rbase, RPS)],
                            out_hbm.at[c, pl.ds(rbase, RPS)])

        @pl.when(s == NS - 1)
        def _tail():
            pltpu.sync_copy(acc.at[pl.ds((NS - 1) * RPS, LASTR)],
                            out_hbm.at[c, pl.ds((NS - 1) * RPS, LASTR)])

    return k(z, src, dst)


def _sc_edgeprep_impl(ea8, dst_p, N, H, ED, CH):
    E = ea8.shape[0] * ea8.shape[1] // ED
    EPR = ea8.shape[1] // ED
    NC, NS, LN = _sc_geometry()
    NW = NC * NS
    NIT = E // (NW * CH)
    EPW = NIT * CH
    XTRA = (E - NW * EPW) // CH
    EMAIN = NW * EPW
    NPAD, RPS, LASTR = _row_partition(N, NS)
    ZR = 8
    mesh = plsc.VectorSubcoreMesh(core_axis_name="c", subcore_axis_name="s")

    NB = 2

    @functools.partial(
        pl.kernel,
        mesh=mesh,
        out_type=jax.ShapeDtypeStruct((NC, N, H), _F32),
        scratch_types=[
            pltpu.VMEM((NB, CH), _I32),
            pltpu.VMEM((NB, CH), _I32),
            pltpu.VMEM((NB, CH // EPR, H), _F32),
            pltpu.VMEM((NB, CH, H), _F32),
            pltpu.VMEM((ZR, H), _F32),
            pltpu.VMEM_SHARED((NPAD, H), _F32),
        ] + [pltpu.SemaphoreType.DMA] * (2 * NB + 1),
    )
    def k(ea_hbm, dst_hbm, aux_out, dst_v, dsc_v, ea_v, rows_v,
          zero_v, acc, *sems):
        semi = sems[0:NB]
        sems_ = sems[NB:2 * NB]
        semz = sems[2 * NB]
        c = lax.axis_index("c")
        s = lax.axis_index("s")
        wid = s * NC + c
        ebase = wid * EPW
        zvec = jnp.zeros((LN,), _F32)
        lane = lax.iota(_I32, LN)
        onehot = jnp.where(lane == 0, jnp.float32(1.0), jnp.float32(0.0))

        def init_zero(i, carry):
            for j in range(H // LN):
                zero_v[i, pl.ds(j * LN, LN)] = zvec
            return carry

        lax.fori_loop(0, ZR, init_zero, 0)

        def init_rows(i, carry):
            for u in range(NB):
                rows_v[u, i, pl.ds(LN, LN)] = onehot
                for j in range(2, H // LN):
                    rows_v[u, i, pl.ds(j * LN, LN)] = zvec
            return carry

        lax.fori_loop(0, CH, init_rows, 0)
        rbase = s * RPS

        def fetch_at(b, u):
            pltpu.async_copy(dst_hbm.at[pl.ds(b, CH)], dst_v.at[u], semi[u])
            r8 = pl.multiple_of(b // EPR, 8)
            pltpu.async_copy(ea_hbm.at[pl.ds(r8, CH // EPR)],
                             ea_v.at[u], semi[u])

        def fetch(i, u):
            fetch_at(ebase + i * CH, u)

        def wait_fetch(u):
            pltpu.make_async_copy(dst_hbm.at[pl.ds(0, CH)], dst_v.at[u],
                                  semi[u]).wait()
            pltpu.make_async_copy(ea_hbm.at[pl.ds(0, CH // EPR)],
                                  ea_v.at[u], semi[u]).wait()

        def wait_scat(u):
            pltpu.make_async_copy(z_dummy.at[pl.ds(0, CH)], rows_v.at[u],
                                  sems_[u]).wait()

        z_dummy = aux_out.at[0]

        for u in range(NB):
            fetch(u, u)

        def zacc(i, carry):
            pltpu.async_copy(zero_v, acc.at[pl.ds(rbase + i * ZR, ZR)], semz)
            return carry

        lax.fori_loop(0, RPS // ZR, zacc, 0)

        def zdrain(i, carry):
            pltpu.make_async_copy(z_dummy.at[pl.ds(0, ZR)], zero_v,
                                  semz).wait()
            return carry

        lax.fori_loop(0, RPS // ZR, zdrain, 0)
        plsc.subcore_barrier()

        def stage(i, u):
            @pl.when(i > 0)
            def _():
                wait_scat(u)

            wait_fetch(u)

            def repack(q, carry2):
                for j in range(EPR):
                    rows_v[u, q * EPR + j, pl.ds(0, LN)] = (
                        ea_v[u, q, pl.ds(j * ED, LN)])
                return carry2

            lax.fori_loop(0, CH // EPR, repack, 0)
            for g in range(CH // LN):
                dsc_v[u, pl.ds(g * LN, LN)] = dst_v[u, pl.ds(g * LN, LN)]
            pltpu.async_copy(rows_v.at[u], acc.at[dsc_v.at[u]], sems_[u],
                             add=True)

        def body(i, carry):
            for u in range(NB):
                idx = i * NB + u
                stage(i, u)

                @pl.when(idx + NB < NIT)
                def _():
                    fetch(idx + NB, u)

            return carry

        lax.fori_loop(0, NIT // NB, body, 0)
        for u in range(NIT - (NIT // NB) * NB):
            stage(NIT // NB, u)
        for u in range(min(NB, NIT)):
            wait_scat(u)
        if XTRA:
            @pl.when(wid < XTRA)
            def _extra():
                fetch_at(EMAIN + wid * CH, 0)
                wait_fetch(0)

                def repack_x(q, carry2):
                    for j in range(EPR):
                        rows_v[0, q * EPR + j, pl.ds(0, LN)] = (
                            ea_v[0, q, pl.ds(j * ED, LN)])
                    return carry2

                lax.fori_loop(0, CH // EPR, repack_x, 0)
                for g in range(CH // LN):
                    dsc_v[0, pl.ds(g * LN, LN)] = dst_v[0, pl.ds(g * LN, LN)]
                pltpu.sync_copy(rows_v.at[0], acc.at[dsc_v.at[0]], add=True)
        plsc.subcore_barrier()

        @pl.when(s < NS - 1)
        def _full():
            pltpu.sync_copy(acc.at[pl.ds(rbase, RPS)],
                            aux_out.at[c, pl.ds(rbase, RPS)])

        @pl.when(s == NS - 1)
        def _tail():
            pltpu.sync_copy(acc.at[pl.ds((NS - 1) * RPS, LASTR)],
                            aux_out.at[c, pl.ds((NS - 1) * RPS, LASTR)])

    return k(ea8, dst_p)


def _dotT(a, w):
    return lax.dot_general(a, w, (((1,), (1,)), ((), ())),
                           preferred_element_type=_F32)


def _tc_auxred(aux):
    NC, N, H = aux.shape

    def body(a_ref, o_ref):
        o_ref[...] = a_ref[0, :, 0:32] + a_ref[1, :, 0:32]

    return pl.pallas_call(
        body, out_shape=jax.ShapeDtypeStruct((N, 32), _F32))(aux)


def _tc_prep(batch2d, x, nW0, vn_emb, Bsz):
    _, N = batch2d.shape
    H = nW0.shape[0]

    def body(batch_ref, x_ref, w_ref, vne_ref, Pn_ref, cnt_ref, z_ref,
             vn_ref):
        iota = lax.broadcasted_iota(_I32, (Bsz, N), 0)
        P = (iota == batch_ref[...]).astype(_F32)
        counts = jnp.sum(P, axis=1, keepdims=True)
        Pn = P / jnp.maximum(counts, 1.0)
        Pn_ref[...] = Pn
        cnt_ref[...] = counts
        vn0 = jnp.broadcast_to(vne_ref[...], (Bsz, H))
        vn_ref[...] = vn0
        z_ref[...] = _dotT(x_ref[...], w_ref[...]) + lax.dot_general(
            Pn, counts * vn0, (((0,), (0,)), ((), ())),
            preferred_element_type=_F32)

    return pl.pallas_call(
        body,
        out_shape=(
            jax.ShapeDtypeStruct((Bsz, N), _F32),
            jax.ShapeDtypeStruct((Bsz, 1), _F32),
            jax.ShapeDtypeStruct((N, H), _F32),
            jax.ShapeDtypeStruct((Bsz, H), _F32),
        ),
    )(batch2d, x, nW0, vn_emb)


def _tc_layer(Ap, ead, vn, Pn, cnt, eW, beff, W1, b1, W2, b2,
              vW1, vb1, vW2, vb2, nW_next):
    NC, N, H = Ap.shape
    Bsz = Pn.shape[0]

    def body(Ap_ref, ead_ref, vn_ref, Pn_ref, cnt_ref, eW_ref, beff_ref,
             W1_ref, b1_ref, W2_ref, b2_ref, vW1_ref, vb1_ref, vW2_ref,
             vb2_ref, nWn_ref, z_ref, vno_ref):
        A = Ap_ref[0] + Ap_ref[1]
        ead = ead_ref[...]
        EA = ead[:, 0:16]
        deg = ead[:, 16:17]
        agg = A + _dotT(EA, eW_ref[...]) + deg * beff_ref[...]
        t = jnp.maximum(_dotT(agg, W1_ref[...]) + b1_ref[...], 0.0)
        h = jnp.maximum(_dotT(t, W2_ref[...]) + b2_ref[...], 0.0)
        pooled = jnp.dot(Pn_ref[...], h, preferred_element_type=_F32)
        q = jnp.maximum(_dotT(pooled, vW1_ref[...]) + vb1_ref[...], 0.0)
        vnu = jnp.maximum(_dotT(q, vW2_ref[...]) + vb2_ref[...], 0.0)
        vn_new = vn_ref[...] + vnu
        vno_ref[...] = vn_new
        z_ref[...] = _dotT(h, nWn_ref[...]) + lax.dot_general(
            Pn_ref[...], cnt_ref[...] * vn_new, (((0,), (0,)), ((), ())),
            preferred_element_type=_F32)

    return pl.pallas_call(
        body,
        out_shape=(
            jax.ShapeDtypeStruct((N, H), _F32),
            jax.ShapeDtypeStruct((Bsz, H), _F32),
        ),
    )(Ap, ead, vn, Pn, cnt, eW, beff, W1, b1, W2, b2, vW1, vb1, vW2, vb2,
      nW_next)


def _tc_last(Ap, ead, Pn, eW, beff, W1, b1, W2, b2, fcW, fcb):
    NC, N, H = Ap.shape
    Bsz = Pn.shape[0]
    OUT = fcW.shape[0]

    def body(Ap_ref, ead_ref, Pn_ref, eW_ref, beff_ref, W1_ref,
             b1_ref, W2_ref, b2_ref, fcW_ref, fcb_ref, o_ref):
        A = Ap_ref[0] + Ap_ref[1]
        ead = ead_ref[...]
        EA = ead[:, 0:16]
        deg = ead[:, 16:17]
        agg = A + _dotT(EA, eW_ref[...]) + deg * beff_ref[...]
        t = jnp.maximum(_dotT(agg, W1_ref[...]) + b1_ref[...], 0.0)
        h = jnp.maximum(_dotT(t, W2_ref[...]) + b2_ref[...], 0.0)
        pooled = jnp.dot(Pn_ref[...], h, preferred_element_type=_F32)
        o_ref[...] = _dotT(pooled, fcW_ref[...]) + fcb_ref[...]

    return pl.pallas_call(
        body,
        out_shape=jax.ShapeDtypeStruct((Bsz, OUT), _F32),
    )(Ap, ead, Pn, eW, beff, W1, b1, W2, b2, fcW, fcb)


def kernel(x, edge_index, edge_attr, batch, node_W, node_b, edge_W, edge_b,
           mlp_W1, mlp_b1, mlp_W2, mlp_b2, vn_emb, vnmlp_W1, vnmlp_b1,
           vnmlp_W2, vnmlp_b2, fc_W, fc_b):
    N, F = x.shape
    L, H, _ = node_W.shape
    Bsz = 128
    CH = 64
    src = edge_index[0]
    dst = edge_index[1]
    r2 = lambda v: v.reshape(1, -1)

    ea8 = edge_attr.reshape(-1, 128)
    aux = _sc_edgeprep_impl(ea8, dst, N, H, edge_attr.shape[1], CH)
    ead = _tc_auxred(aux)
    Pn, cnt, z, vn = _tc_prep(batch.reshape(1, N), x, node_W[0], vn_emb, Bsz)

    out = None
    for l in range(L):
        Ap = _sc_segsum(z, src, dst, CH)
        beff = r2(node_b[l] + edge_b[l])
        if l + 1 < L:
            z, vn = _tc_layer(Ap, ead, vn, Pn, cnt, edge_W[l], beff,
                              mlp_W1[l], r2(mlp_b1[l]), mlp_W2[l],
                              r2(mlp_b2[l]), vnmlp_W1, r2(vnmlp_b1),
                              vnmlp_W2, r2(vnmlp_b2), node_W[l + 1])
        else:
            out = _tc_last(Ap, ead, Pn, edge_W[l], beff, mlp_W1[l],
                           r2(mlp_b1[l]), mlp_W2[l], r2(mlp_b2[l]), fc_W,
                           r2(fc_b))
    return out
